# baseline (speedup 1.0000x reference)
import jax, jax.numpy as jnp
import math
from jax.experimental import pallas as pl

_N = 10000; _E = 160000; _G = 20
_RATIO = 0.5; _K = 3; _JA = 1.0; _JB = 1.0


def _adj_h(ei, emask, n):
    loop = jnp.arange(n)
    src = jnp.concatenate([ei[0], loop]); dst = jnp.concatenate([ei[1], loop])
    m = jnp.concatenate([emask, jnp.ones((n,), emask.dtype)])
    deg = jnp.zeros((n,), jnp.float32).at[dst].add(m)
    dinv = 1.0 / jnp.sqrt(jnp.clip(deg, 1.0))
    coef = dinv[src] * dinv[dst] * m
    return src, dst, coef


def _gcn_h(x, ei, emask, W, b):
    n = x.shape[0]
    h = x @ W
    src, dst, coef = _adj_h(ei, emask, n)
    out = jnp.zeros_like(h).at[dst].add(coef[:, None] * h[src])
    return out + b


def _jfit_h(s, ei, emask, coefs):
    n = s.shape[0]
    src, dst, coef = _adj_h(ei, emask, n)
    def Av(v):
        return jnp.zeros((n,), v.dtype).at[dst].add(coef * v[src])
    a, b = _JA, _JB
    P0 = s
    out = coefs[0] * P0
    P1 = (a - b) / 2.0 * s + (a + b + 2.0) / 2.0 * Av(s)
    out = out + coefs[1] * P1
    Pm2, Pm1 = P0, P1
    for k in range(2, _K + 1):
        c1 = 2.0 * k * (k + a + b) * (2 * k + a + b - 2)
        c2 = (2 * k + a + b - 1) * (a * a - b * b)
        c3 = (2 * k + a + b - 2) * (2 * k + a + b - 1) * (2 * k + a + b)
        c4 = 2.0 * (k + a - 1) * (k + b - 1) * (2 * k + a + b)
        Pk = (c2 * Pm1 + c3 * Av(Pm1) - c4 * Pm2) / c1
        out = out + coefs[k] * Pk
        Pm2, Pm1 = Pm1, Pk
    return out


def _jpool_h(x, ei, emask, batch_arr, w, coefs):
    n = x.shape[0]
    s = x @ w
    fit = jnp.tanh(_jfit_h(s, ei, emask, coefs))
    counts = jax.ops.segment_sum(jnp.ones((n,), jnp.int32), batch_arr, num_segments=_G)
    kk = jnp.where(counts > 0,
                   jnp.maximum(1, jnp.ceil(_RATIO * counts.astype(jnp.float32)).astype(jnp.int32)),
                   0)
    o1 = jnp.argsort(-fit, stable=True)
    order = o1[jnp.argsort(batch_arr[o1], stable=True)]
    gs = batch_arr[order]
    gsafe = jnp.minimum(gs, _G - 1)
    offsets = jnp.cumsum(counts) - counts
    rank = jnp.arange(n, dtype=jnp.int32) - offsets[gsafe]
    keep_n = (gs < _G) & (rank < kk[gsafe])
    m = jnp.sum(kk)
    perm = order[jnp.argsort(jnp.logical_not(keep_n), stable=True)]
    posmask = jnp.arange(n, dtype=jnp.int32) < m
    x_new = jnp.where(posmask[:, None], x[perm] * fit[perm][:, None], 0.0)
    batch_new = jnp.where(posmask, batch_arr[perm], _G).astype(jnp.int32)
    newidx = jnp.where(posmask, jnp.arange(n, dtype=jnp.int32), -1)
    mapping = jnp.full((n,), -1, jnp.int32).at[perm].set(newidx)
    sm = mapping[ei[0]]; dm = mapping[ei[1]]
    keep = (sm >= 0) & (dm >= 0) & (emask > 0)
    ei_new = jnp.stack([jnp.where(keep, sm, 0), jnp.where(keep, dm, 0)])
    return x_new, ei_new, keep.astype(jnp.float32), batch_new


def _readout_h(x, batch_arr):
    seg = jnp.asarray(batch_arr)
    cnt = jax.ops.segment_sum(jnp.ones((x.shape[0],), jnp.float32), seg, num_segments=_G)
    mean = jax.ops.segment_sum(x, seg, num_segments=_G) / jnp.clip(cnt, 1.0)[:, None]
    mx = jax.ops.segment_max(x, seg, num_segments=_G)
    mx = jnp.where(cnt[:, None] > 0, mx, 0.0)
    return jnp.concatenate([mx, mean], axis=1)


def _bnorm_h(x, g, b):
    mu = jnp.mean(x, axis=0)
    var = jnp.var(x, axis=0)
    return g * (x - mu) / jnp.sqrt(var + 1e-5) + b


def _bnorm_m_h(x, g, b, rowmask):
    m = jnp.sum(rowmask)
    wm = rowmask[:, None]
    mu = jnp.sum(x * wm, axis=0) / m
    var = jnp.sum(wm * (x - mu) ** 2, axis=0) / m
    return g * (x - mu) / jnp.sqrt(var + 1e-5) + b


def _ident_body(x_ref, o_ref):
    o_ref[...] = x_ref[...]


def _pl_ident(x):
    return pl.pallas_call(
        _ident_body,
        out_shape=jax.ShapeDtypeStruct(x.shape, x.dtype),
    )(x)


def kernel(x, edge_index, batch, W1, b1, bn1_g, bn1_b, pool1_w, pool1_c, W2, b2, bn2_g, bn2_b, pool2_w, pool2_c, W3, b3, bn3_g, bn3_b, pool3_w, pool3_c, lin1_W, lin1_b, bn4_g, bn4_b, lin2_W, lin2_b, bn5_g, bn5_b, lin3_W, lin3_b):
    batch_arr = jnp.asarray(batch).astype(jnp.int32)
    emask = jnp.ones((edge_index.shape[1],), jnp.float32)
    h = jax.nn.relu(_bnorm_h(_gcn_h(x, edge_index, emask, W1, b1), bn1_g, bn1_b))
    h, ei, emask, batch_arr = _jpool_h(h, edge_index, emask, batch_arr, pool1_w, pool1_c)
    x1 = _readout_h(h, batch_arr)
    nmask = (batch_arr < _G).astype(jnp.float32)
    h = jax.nn.relu(_bnorm_m_h(_gcn_h(h, ei, emask, W2, b2), bn2_g, bn2_b, nmask))
    h, ei, emask, batch_arr = _jpool_h(h, ei, emask, batch_arr, pool2_w, pool2_c)
    x2 = _readout_h(h, batch_arr)
    nmask = (batch_arr < _G).astype(jnp.float32)
    h = jax.nn.relu(_bnorm_m_h(_gcn_h(h, ei, emask, W3, b3), bn3_g, bn3_b, nmask))
    h, ei, emask, batch_arr = _jpool_h(h, ei, emask, batch_arr, pool3_w, pool3_c)
    x3 = _readout_h(h, batch_arr)
    z = (x1 + x2 + x3) / 3.0
    z = jax.nn.relu(_bnorm_h(z @ lin1_W + lin1_b, bn4_g, bn4_b))
    z = jax.nn.relu(_bnorm_h(z @ lin2_W + lin2_b, bn5_g, bn5_b))
    logits = z @ lin3_W + lin3_b
    return _pl_ident(jax.nn.log_softmax(logits, axis=-1))


# speedup vs baseline: 5.9850x; 5.9850x over previous
import functools
import math

import jax
import jax.numpy as jnp
from jax import lax
from jax.experimental import pallas as pl
from jax.experimental.pallas import tpu as pltpu
from jax.experimental.pallas import tpu_sc as plsc

NG = 20
NGP = 32
NREAL = 10000
EREAL = 160000
MP = (10240, 6144, 4096, 2048)
E2 = 161792
NT = 16
F32 = jnp.float32
I32 = jnp.int32


def _mm_pre_call(mp, kd, use_scale):
    r = 512

    def body(*refs):
        if use_scale:
            x_ref, w_ref, dinv_ref, rs_ref, hpre_ref, hq_ref = refs
            a = x_ref[...] * rs_ref[...]
        else:
            x_ref, w_ref, dinv_ref, hpre_ref, hq_ref = refs
            a = x_ref[...]
        acc = jnp.dot(a, w_ref[...], preferred_element_type=F32)
        hpre_ref[...] = acc
        scaled = acc * dinv_ref[...]
        for q in range(4):
            hq_ref[q] = scaled[:, q * 128:(q + 1) * 128]

    in_specs = [
        pl.BlockSpec((r, kd), lambda i: (i, 0)),
        pl.BlockSpec((kd, 512), lambda i: (0, 0)),
        pl.BlockSpec((r, 1), lambda i: (i, 0)),
    ]
    if use_scale:
        in_specs.append(pl.BlockSpec((r, 1), lambda i: (i, 0)))
    return pl.pallas_call(
        body,
        grid=(mp // r,),
        in_specs=in_specs,
        out_specs=[
            pl.BlockSpec((r, 512), lambda i: (i, 0)),
            pl.BlockSpec((4, r, 128), lambda i: (0, i, 0)),
        ],
        out_shape=[
            jax.ShapeDtypeStruct((mp, 512), F32),
            jax.ShapeDtypeStruct((4, mp, 128), F32),
        ],
    )


def _bnormA_call(mp):
    r = 512

    def body(s_ref, hpre_ref, dinv_ref, batch_ref, bias_ref, z_ref, st_ref):
        i = pl.program_id(0)
        mask = (batch_ref[...] < NG).astype(F32)
        dv = dinv_ref[...]
        sc = jnp.concatenate([s_ref[0], s_ref[1], s_ref[2], s_ref[3]], axis=1)
        zv = dv * sc + dv * dv * hpre_ref[...] + bias_ref[...]
        z_ref[...] = zv

        @pl.when(i == 0)
        def _():
            st_ref[...] = jnp.zeros_like(st_ref)

        st_ref[0:1, :] += jnp.sum(zv * mask, axis=0, keepdims=True)
        st_ref[1:2, :] += jnp.sum(zv * zv * mask, axis=0, keepdims=True)
        st_ref[2:3, :] += jnp.sum(mask) * jnp.ones((1, 512), F32)

    return pl.pallas_call(
        body,
        grid=(mp // r,),
        in_specs=[
            pl.BlockSpec((4, r, 128), lambda i: (0, i, 0)),
            pl.BlockSpec((r, 512), lambda i: (i, 0)),
            pl.BlockSpec((r, 1), lambda i: (i, 0)),
            pl.BlockSpec((r, 1), lambda i: (i, 0)),
            pl.BlockSpec((1, 512), lambda i: (0, 0)),
        ],
        out_specs=[
            pl.BlockSpec((r, 512), lambda i: (i, 0)),
            pl.BlockSpec((8, 512), lambda i: (0, 0)),
        ],
        out_shape=[
            jax.ShapeDtypeStruct((mp, 512), F32),
            jax.ShapeDtypeStruct((8, 512), F32),
        ],
    )


def _bnormB_call(mp):
    r = 512

    def body(z_ref, st_ref, g_ref, b_ref, w_ref, h_ref, s_ref):
        m = st_ref[2, 0]
        mu = st_ref[0:1, :] / m
        var = st_ref[1:2, :] / m - mu * mu
        hv = jnp.maximum(
            g_ref[...] * (z_ref[...] - mu) * lax.rsqrt(var + 1e-5) + b_ref[...],
            0.0)
        h_ref[...] = hv
        s_ref[...] = jnp.dot(hv, w_ref[...])

    return pl.pallas_call(
        body,
        grid=(mp // r,),
        in_specs=[
            pl.BlockSpec((r, 512), lambda i: (i, 0)),
            pl.BlockSpec((8, 512), lambda i: (0, 0)),
            pl.BlockSpec((1, 512), lambda i: (0, 0)),
            pl.BlockSpec((1, 512), lambda i: (0, 0)),
            pl.BlockSpec((512, 1), lambda i: (0, 0)),
        ],
        out_specs=[
            pl.BlockSpec((r, 512), lambda i: (i, 0)),
            pl.BlockSpec((r, 1), lambda i: (i, 0)),
        ],
        out_shape=[
            jax.ShapeDtypeStruct((mp, 512), F32),
            jax.ShapeDtypeStruct((mp, 1), F32),
        ],
    )


def _tanh_key_call(mp):
    r = 512

    def body(f_ref, fit_ref, key_ref):
        fit = jnp.tanh(f_ref[...])
        fit_ref[...] = fit
        u = lax.bitcast_convert_type(fit, I32)
        key_ref[...] = jnp.where(u >= 0, u, u ^ 0x7FFFFFFF)

    return pl.pallas_call(
        body,
        grid=(mp // r,),
        in_specs=[pl.BlockSpec((r, 1), lambda i: (i, 0))],
        out_specs=[pl.BlockSpec((r, 1), lambda i: (i, 0)),
                   pl.BlockSpec((r, 1), lambda i: (i, 0))],
        out_shape=[jax.ShapeDtypeStruct((mp, 1), F32),
                   jax.ShapeDtypeStruct((mp, 1), I32)],
    )


def _dinv_call(mp):
    def body(deg_ref, d2_ref, d1_ref):
        dv = lax.rsqrt(jnp.clip(deg_ref[...] + 1.0, 1.0))
        d2_ref[...] = dv
        d1_ref[...] = dv

    return pl.pallas_call(
        body,
        grid=(1,),
        in_specs=[pl.BlockSpec((mp, 1), lambda i: (0, 0))],
        out_specs=[pl.BlockSpec((mp, 1), lambda i: (0, 0)),
                   pl.BlockSpec((mp, 1), lambda i: (0, 0))],
        out_shape=[jax.ShapeDtypeStruct((mp, 1), F32),
                   jax.ShapeDtypeStruct((mp, 1), F32)],
    )


def _counts_call(mp):
    r = 512

    def body(b_ref, c_ref):
        i = pl.program_id(0)

        @pl.when(i == 0)
        def _():
            c_ref[...] = jnp.zeros_like(c_ref)

        gi = lax.broadcasted_iota(I32, (1, NGP), 1)
        oh = (b_ref[...] == gi).astype(I32)
        c_ref[...] += jnp.sum(oh, axis=0, keepdims=True)

    return pl.pallas_call(
        body,
        grid=(mp // r,),
        in_specs=[pl.BlockSpec((r, 1), lambda i: (i, 0))],
        out_specs=pl.BlockSpec((1, NGP), lambda i: (0, 0)),
        out_shape=jax.ShapeDtypeStruct((1, NGP), I32),
    )


def _kk_call():
    def body(c_ref, k_ref):
        c = c_ref[...]
        gi = lax.broadcasted_iota(I32, (1, NGP), 1)
        k_ref[...] = jnp.where((c > 0) & (gi < NG),
                               jnp.maximum(1, (c + 1) // 2), 0)

    return pl.pallas_call(
        body,
        grid=(1,),
        in_specs=[pl.BlockSpec((1, NGP), lambda i: (0, 0))],
        out_specs=pl.BlockSpec((1, NGP), lambda i: (0, 0)),
        out_shape=jax.ShapeDtypeStruct((1, NGP), I32),
    )


def _rank_call(mp):
    ri, rj = 256, 512

    def body(kc_ref, bc_ref, kr_ref, br_ref, o_ref):
        i = pl.program_id(0)
        j = pl.program_id(1)

        @pl.when(j == 0)
        def _():
            o_ref[...] = jnp.zeros_like(o_ref)

        ii = i * ri + lax.broadcasted_iota(I32, (ri, 1), 0)
        jj = j * rj + lax.broadcasted_iota(I32, (1, rj), 1)
        kc = kc_ref[...]
        kr = kr_ref[...]
        same = bc_ref[...] == br_ref[...]
        cmp = (kr > kc) | ((kr == kc) & (jj < ii))
        o_ref[...] += jnp.sum((same & cmp).astype(I32), axis=1, keepdims=True)

    return pl.pallas_call(
        body,
        grid=(mp // ri, mp // rj),
        in_specs=[
            pl.BlockSpec((ri, 1), lambda i, j: (i, 0)),
            pl.BlockSpec((ri, 1), lambda i, j: (i, 0)),
            pl.BlockSpec((1, rj), lambda i, j: (0, j)),
            pl.BlockSpec((1, rj), lambda i, j: (0, j)),
        ],
        out_specs=pl.BlockSpec((ri, 1), lambda i, j: (i, 0)),
        out_shape=jax.ShapeDtypeStruct((mp, 1), I32),
    )


def _mapping_call(mp, mn):
    r = 512
    trash0 = mn - 128

    def body(rank_ref, b_ref, fit_ref, kk_ref, midx_ref, map_ref, bv_ref,
             fv_ref):
        i = pl.program_id(0)
        gi = lax.broadcasted_iota(I32, (1, NGP), 1)
        kkv = kk_ref[...]
        b = b_ref[...]
        rank = rank_ref[...]
        kk_i = jnp.sum(jnp.where(b == gi, kkv, 0), axis=1, keepdims=True)
        off_i = jnp.sum(jnp.where(gi < b, kkv, 0), axis=1, keepdims=True)
        keep = (rank < kk_i) & (b < NG)
        mapping = jnp.where(keep, off_i + rank, -1)
        map_ref[...] = mapping
        rowid = i * r + lax.broadcasted_iota(I32, (r, 1), 0)
        midx_ref[...] = jnp.where(keep, mapping, trash0 + (rowid & 127))
        bv_ref[...] = jnp.where(keep, b, NG)
        fv_ref[...] = jnp.where(keep, fit_ref[...], 0.0)

    return pl.pallas_call(
        body,
        grid=(mp // r,),
        in_specs=[
            pl.BlockSpec((r, 1), lambda i: (i, 0)),
            pl.BlockSpec((r, 1), lambda i: (i, 0)),
            pl.BlockSpec((r, 1), lambda i: (i, 0)),
            pl.BlockSpec((1, NGP), lambda i: (0, 0)),
        ],
        out_specs=[pl.BlockSpec((r, 1), lambda i: (i, 0))] * 4,
        out_shape=[jax.ShapeDtypeStruct((mp, 1), I32),
                   jax.ShapeDtypeStruct((mp, 1), I32),
                   jax.ShapeDtypeStruct((mp, 1), I32),
                   jax.ShapeDtypeStruct((mp, 1), F32)],
    )


def _readout_call(mp):
    r = 512

    def body(x_ref, f_ref, b_ref, sum_ref, mx_ref):
        i = pl.program_id(0)
        xv = x_ref[...] * f_ref[...]
        b = b_ref[...]
        gi = lax.broadcasted_iota(I32, (1, NGP), 1)
        oh = (b == gi).astype(F32)

        @pl.when(i == 0)
        def _():
            sum_ref[...] = jnp.zeros_like(sum_ref)
            mx_ref[...] = jnp.full_like(mx_ref, -jnp.inf)

        sum_ref[...] += lax.dot_general(oh, xv, (((0,), (0,)), ((), ())),
                                        preferred_element_type=F32)
        for g in range(NG):
            row = jnp.max(jnp.where(b == g, xv, -jnp.inf), axis=0,
                          keepdims=True)
            mx_ref[g:g + 1, :] = jnp.maximum(mx_ref[g:g + 1, :], row)

    return pl.pallas_call(
        body,
        grid=(mp // r,),
        in_specs=[
            pl.BlockSpec((r, 512), lambda i: (i, 0)),
            pl.BlockSpec((r, 1), lambda i: (i, 0)),
            pl.BlockSpec((r, 1), lambda i: (i, 0)),
        ],
        out_specs=[pl.BlockSpec((NGP, 512), lambda i: (0, 0)),
                   pl.BlockSpec((NGP, 512), lambda i: (0, 0))],
        out_shape=[jax.ShapeDtypeStruct((NGP, 512), F32),
                   jax.ShapeDtypeStruct((NGP, 512), F32)],
    )


def _head_call():
    def bn(t, g, b, rm):
        mu = jnp.sum(t * rm, axis=0, keepdims=True) / float(NG)
        var = jnp.sum(rm * (t - mu) ** 2, axis=0, keepdims=True) / float(NG)
        return jnp.maximum(g * (t - mu) * lax.rsqrt(var + 1e-5) + b, 0.0)

    def body(s1, m1, k1, s2, m2, k2, s3, m3, k3, wa, wb, l1b, g4, b4, w2, l2b,
             g5, b5, w3, l3b, o_ref):
        rm = (lax.broadcasted_iota(I32, (NGP, 1), 0) < NG).astype(F32)

        def xl(s_ref, m_ref, k_ref):
            kc = k_ref[...].astype(F32)
            mean = s_ref[...] / jnp.clip(kc, 1.0)
            mx = jnp.where(kc > 0, m_ref[...], 0.0)
            return mx, mean

        x1m, x1a = xl(s1, m1, k1)
        x2m, x2a = xl(s2, m2, k2)
        x3m, x3a = xl(s3, m3, k3)
        mxz = (x1m + x2m + x3m) / 3.0
        mnz = (x1a + x2a + x3a) / 3.0
        t = (jnp.dot(mxz, wa[...], preferred_element_type=F32)
             + jnp.dot(mnz, wb[...], preferred_element_type=F32) + l1b[...])
        t = bn(t, g4[...], b4[...], rm)
        u = jnp.dot(t, w2[...], preferred_element_type=F32) + l2b[...]
        u = bn(u, g5[...], b5[...], rm)
        logits = jnp.dot(u, w3[...], preferred_element_type=F32) + l3b[...]
        lanem = lax.broadcasted_iota(I32, (1, 128), 1) < 10
        mm = jnp.max(jnp.where(lanem, logits, -jnp.inf), axis=1, keepdims=True)
        lse = jnp.log(jnp.sum(jnp.where(lanem, jnp.exp(logits - mm), 0.0),
                              axis=1, keepdims=True)) + mm
        o_ref[...] = logits - lse

    full = lambda shape: pl.BlockSpec(shape, lambda: tuple(0 for _ in shape))
    in_specs = []
    for _ in range(3):
        in_specs += [full((NGP, 512)), full((NGP, 512)), full((NGP, 1))]
    in_specs += [full((512, 512)), full((512, 512)), full((1, 512)),
                 full((1, 512)), full((1, 512)), full((512, 256)),
                 full((1, 256)), full((1, 256)), full((1, 256)),
                 full((256, 128)), full((1, 128))]
    return pl.pallas_call(
        body,
        grid=(),
        in_specs=in_specs,
        out_specs=full((NGP, 128)),
        out_shape=jax.ShapeDtypeStruct((NGP, 128), F32),
    )


def _vfill(ref, n, val, dtype):
    for k in range(n // 16):
        ref[pl.ds(k * 16, 16)] = jnp.full((16,), val, dtype)


_MESH = None


def _mesh():
    global _MESH
    if _MESH is None:
        _MESH = plsc.VectorSubcoreMesh(core_axis_name="c", subcore_axis_name="s")
    return _MESH


def _sck_deg_call(mp):
    sl = mp // NT
    ept = E2 // NT
    nb = ept // 128

    @functools.partial(
        pl.kernel,
        out_type=jax.ShapeDtypeStruct((mp,), F32),
        mesh=_mesh(),
        scratch_types=[
            pltpu.VMEM((128,), I32),
            pltpu.VMEM((128,), F32),
            pltpu.VMEM_SHARED((mp,), F32),
        ],
    )
    def k(dst_hbm, zeros_hbm, out_hbm, idxv, onesv, acc):
        c = lax.axis_index("c")
        s = lax.axis_index("s")

        @pl.when(c == 0)
        def _():
            pltpu.sync_copy(zeros_hbm.at[pl.ds(s * sl, sl)],
                            acc.at[pl.ds(s * sl, sl)])
            _vfill(onesv, 128, 1.0, F32)
            plsc.subcore_barrier()

            def blk(b, carry):
                base = s * ept + b * 128
                pltpu.sync_copy(dst_hbm.at[pl.ds(base, 128)], idxv)
                pltpu.sync_copy(onesv, acc.at[idxv], add=True)
                return carry

            lax.fori_loop(0, nb, blk, 0)
            plsc.subcore_barrier()
            pltpu.sync_copy(acc.at[pl.ds(s * sl, sl)],
                            out_hbm.at[pl.ds(s * sl, sl)])

    return k


def _sck_conv_call(mp):
    sl = mp // NT
    ept = E2 // NT
    nb = ept // 128

    @functools.partial(
        pl.kernel,
        out_type=jax.ShapeDtypeStruct((4, mp, 128), F32),
        mesh=_mesh(),
        scratch_types=[
            pltpu.VMEM((128,), I32),
            pltpu.VMEM((128,), I32),
            pltpu.VMEM((128, 128), F32),
            pltpu.VMEM_SHARED((mp, 128), F32),
            pltpu.SemaphoreType.DMA,
        ],
    )
    def k(hq_hbm, src_hbm, dst_hbm, zeros_hbm, out_hbm, sidx, didx, rows, acc,
          sem):
        c = lax.axis_index("c")
        s = lax.axis_index("s")

        def one_chunk(q):
            pltpu.sync_copy(zeros_hbm.at[pl.ds(s * sl, sl)],
                            acc.at[pl.ds(s * sl, sl)])
            plsc.subcore_barrier()

            def blk(b, carry):
                base = s * ept + b * 128
                pltpu.sync_copy(src_hbm.at[pl.ds(base, 128)], sidx)
                pltpu.sync_copy(dst_hbm.at[pl.ds(base, 128)], didx)
                pltpu.async_copy(hq_hbm.at[q].at[sidx], rows, sem).wait()
                pltpu.sync_copy(rows, acc.at[didx], add=True)
                return carry

            lax.fori_loop(0, nb, blk, 0)
            plsc.subcore_barrier()
            pltpu.sync_copy(acc.at[pl.ds(s * sl, sl)],
                            out_hbm.at[q].at[pl.ds(s * sl, sl)])
            plsc.subcore_barrier()

        for ci in range(2):
            @pl.when(c == ci)
            def _():
                one_chunk(2 * ci)
                one_chunk(2 * ci + 1)

    return k


def _sck_fitness_call(mp, acoefs):
    sl = mp // NT
    ept = E2 // NT
    nb = ept // 128

    @functools.partial(
        pl.kernel,
        out_type=jax.ShapeDtypeStruct((mp,), F32),
        mesh=_mesh(),
        scratch_types=[
            pltpu.VMEM((128,), I32),
            pltpu.VMEM((128,), I32),
            pltpu.VMEM((128,), F32),
            pltpu.VMEM((sl,), F32),
            pltpu.VMEM((sl,), F32),
            pltpu.VMEM((sl,), F32),
            pltpu.VMEM((sl,), F32),
            pltpu.VMEM((sl,), F32),
            pltpu.VMEM((sl,), F32),
            pltpu.VMEM((64,), F32),
            pltpu.VMEM_SHARED((mp,), F32),
            pltpu.VMEM_SHARED((mp,), F32),
            pltpu.SemaphoreType.DMA,
        ],
    )
    def k(s_hbm, dinv_hbm, src_hbm, dst_hbm, coefb_hbm, zeros_hbm, out_hbm,
          sidx, didx, vals, dsl, stage, pm2, pm1, pk, osl, cvec, upub, acc,
          sem):
        c = lax.axis_index("c")
        s = lax.axis_index("s")

        @pl.when(c == 0)
        def _():
            base = s * sl
            pltpu.sync_copy(dinv_hbm.at[pl.ds(base, sl)], dsl)
            pltpu.sync_copy(s_hbm.at[pl.ds(base, sl)], pm2)
            pltpu.sync_copy(coefb_hbm, cvec)

            def coef(kc):
                return cvec[pl.ds(kc * 16, 16)]

            def spmv(v_ref):
                for t in range(sl // 16):
                    d16 = pl.ds(t * 16, 16)
                    stage[d16] = dsl[d16] * v_ref[d16]
                pltpu.sync_copy(stage, upub.at[pl.ds(base, sl)])
                pltpu.sync_copy(zeros_hbm.at[pl.ds(base, sl)],
                                acc.at[pl.ds(base, sl)])
                plsc.subcore_barrier()

                def blk(b, carry):
                    eb = s * ept + b * 128
                    pltpu.sync_copy(src_hbm.at[pl.ds(eb, 128)], sidx)
                    pltpu.sync_copy(dst_hbm.at[pl.ds(eb, 128)], didx)
                    pltpu.async_copy(upub.at[sidx], vals, sem).wait()
                    pltpu.sync_copy(vals, acc.at[didx], add=True)
                    return carry

                lax.fori_loop(0, nb, blk, 0)
                plsc.subcore_barrier()
                pltpu.sync_copy(acc.at[pl.ds(base, sl)], stage)

            c0 = coef(0)
            for t in range(sl // 16):
                d16 = pl.ds(t * 16, 16)
                osl[d16] = c0 * pm2[d16]
            spmv(pm2)
            c1c = coef(1)
            for t in range(sl // 16):
                d16 = pl.ds(t * 16, 16)
                av = dsl[d16] * stage[d16] + dsl[d16] * dsl[d16] * pm2[d16]
                pm1[d16] = 2.0 * av
                osl[d16] = osl[d16] + c1c * pm1[d16]
            for kk_, (cc1, cc2, cc3, cc4) in enumerate(acoefs):
                spmv(pm1)
                ck = coef(kk_ + 2)
                for t in range(sl // 16):
                    d16 = pl.ds(t * 16, 16)
                    av = dsl[d16] * stage[d16] + dsl[d16] * dsl[d16] * pm1[d16]
                    pkv = (cc2 * pm1[d16] + cc3 * av - cc4 * pm2[d16]) / cc1
                    pk[d16] = pkv
                    osl[d16] = osl[d16] + ck * pkv
                for t in range(sl // 16):
                    d16 = pl.ds(t * 16, 16)
                    pm2[d16] = pm1[d16]
                    pm1[d16] = pk[d16]
            pltpu.sync_copy(osl, out_hbm.at[pl.ds(base, sl)])

    return k


def _sck_pool_call(mp, mn, phases=(0, 1, 2, 3)):
    sl = mp // NT
    sln = mn // NT
    ept = E2 // NT
    nb = ept // 128
    trash0 = mn - 128

    out_type = [
        jax.ShapeDtypeStruct((mn,), I32),
        jax.ShapeDtypeStruct((mn,), I32),
        jax.ShapeDtypeStruct((mn,), F32),
        jax.ShapeDtypeStruct((mn, 512), F32),
        jax.ShapeDtypeStruct((E2,), I32),
        jax.ShapeDtypeStruct((E2,), I32),
        jax.ShapeDtypeStruct((mn,), F32),
    ]

    @functools.partial(
        pl.kernel,
        out_type=out_type,
        mesh=_mesh(),
        scratch_types=[
            pltpu.VMEM((128,), I32),
            pltpu.VMEM((128,), I32),
            pltpu.VMEM((128,), I32),
            pltpu.VMEM((128,), F32),
            pltpu.VMEM((128,), F32),
            pltpu.VMEM((128,), I32),
            pltpu.VMEM((128,), F32),
            pltpu.VMEM((64,), I32),
            pltpu.VMEM((64, 512), F32),
            pltpu.VMEM((128,), I32),
            pltpu.VMEM((128,), I32),
            pltpu.VMEM_SHARED((mn,), F32),
            pltpu.SemaphoreType.DMA,
        ],
    )
    def k(map_hbm, midx_hbm, bval_hbm, fval_hbm, iota_hbm, h_hbm, src_hbm,
          dst_hbm, zeros_hbm, minv_hbm, bnew_hbm, fnew_hbm, xraw_hbm,
          srcn_hbm, dstn_hbm, deg_hbm, ia, ib, iv, fv, ones, dbi, dbf, ri,
          rows, smv, dmv, dacc, sem):
        c = lax.axis_index("c")
        s = lax.axis_index("s")

        @pl.when(c == 0)
        def _():
            if 0 not in phases:
                return
            _vfill(dbi, 128, NG, I32)
            _vfill(dbf, 128, 0.0, F32)
            for bidx in range(sln // 128):
                b0 = s * sln + bidx * 128
                pltpu.sync_copy(dbf, fnew_hbm.at[pl.ds(b0, 128)])
                pltpu.sync_copy(dbi, bnew_hbm.at[pl.ds(b0, 128)])
            _vfill(dbi, 128, 0, I32)
            for bidx in range(sln // 128):
                b0 = s * sln + bidx * 128
                pltpu.sync_copy(dbi, minv_hbm.at[pl.ds(b0, 128)])
            plsc.subcore_barrier()

            if 1 not in phases:
                return
            nbk = mp // 128

            def scat(b0):
                pltpu.sync_copy(midx_hbm.at[pl.ds(b0, 128)], ia)
                pltpu.sync_copy(iota_hbm.at[pl.ds(b0, 128)], iv)
                pltpu.sync_copy(iv, minv_hbm.at[ia])
                pltpu.sync_copy(bval_hbm.at[pl.ds(b0, 128)], iv)
                pltpu.sync_copy(iv, bnew_hbm.at[ia])
                pltpu.sync_copy(fval_hbm.at[pl.ds(b0, 128)], fv)
                pltpu.sync_copy(fv, fnew_hbm.at[ia])

            for j in range((nbk + NT - 1) // NT):
                bi = s + j * NT
                if (j + 1) * NT <= nbk:
                    scat(bi * 128)
                else:
                    @pl.when(bi < nbk)
                    def _():
                        scat(bi * 128)
            plsc.subcore_barrier()

            if 2 not in phases:
                return

            def gat(r0):
                pltpu.sync_copy(minv_hbm.at[pl.ds(r0, 64)], ri)
                pltpu.async_copy(h_hbm.at[ri], rows, sem).wait()
                pltpu.sync_copy(rows, xraw_hbm.at[pl.ds(r0, 64)])

            for bidx in range(sln // 64):
                gat(s * sln + bidx * 64)

        @pl.when(c == 1)
        def _():
            if 3 not in phases:
                return
            _vfill(ones, 128, 1.0, F32)
            pltpu.sync_copy(zeros_hbm.at[pl.ds(s * sln, sln)],
                            dacc.at[pl.ds(s * sln, sln)])
            plsc.subcore_barrier()
            io16 = lax.iota(I32, 16)

            def blk(b, carry):
                eb = s * ept + b * 128
                pltpu.sync_copy(src_hbm.at[pl.ds(eb, 128)], ia)
                pltpu.sync_copy(dst_hbm.at[pl.ds(eb, 128)], ib)
                pltpu.async_copy(map_hbm.at[ia], smv, sem).wait()
                pltpu.async_copy(map_hbm.at[ib], dmv, sem).wait()
                for t in range(8):
                    d16 = pl.ds(t * 16, 16)
                    sm = smv[d16]
                    dm = dmv[d16]
                    keep = (sm >= 0) & (dm >= 0)
                    eidx = eb + t * 16 + io16
                    ia[d16] = jnp.where(keep, sm, eidx & 1023)
                    ib[d16] = jnp.where(keep, dm, trash0 + (eidx & 127))
                pltpu.sync_copy(ia, srcn_hbm.at[pl.ds(eb, 128)])
                pltpu.sync_copy(ib, dstn_hbm.at[pl.ds(eb, 128)])
                pltpu.sync_copy(ones, dacc.at[ib], add=True)
                return carry

            lax.fori_loop(0, nb, blk, 0)
            plsc.subcore_barrier()
            pltpu.sync_copy(dacc.at[pl.ds(s * sln, sln)],
                            deg_hbm.at[pl.ds(s * sln, sln)])

    return k


def _jacobi_recur_consts():
    a, b = 1.0, 1.0
    out = []
    for k in range(2, 4):
        c1 = 2.0 * k * (k + a + b) * (2 * k + a + b - 2)
        c2 = (2 * k + a + b - 1) * (a * a - b * b)
        c3 = (2 * k + a + b - 2) * (2 * k + a + b - 1) * (2 * k + a + b)
        c4 = 2.0 * (k + a - 1) * (k + b - 1) * (2 * k + a + b)
        out.append((c1, c2, c3, c4))
    return tuple(out)


def _layer(li, x, rowscale, batch2d, src, dst, deg1d, W, bias, bng, bnb,
           poolw, poolc, counts, zeros1d, zeros2d):
    mp, mn = MP[li], MP[li + 1]
    deg2d = deg1d.reshape(mp, 1)
    dinv2d, dinv1d2 = _dinv_call(mp)(deg2d)
    dinv1d = dinv1d2.reshape(mp)
    if rowscale is None:
        hpre, hq = _mm_pre_call(mp, W.shape[0], False)(x, W, dinv2d)
    else:
        hpre, hq = _mm_pre_call(mp, W.shape[0], True)(x, W, dinv2d, rowscale)
    S = _sck_conv_call(mp)(hq, src, dst, zeros2d[:mp])
    z, st = _bnormA_call(mp)(S, hpre, dinv2d, batch2d, bias.reshape(1, 512))
    h, s2d = _bnormB_call(mp)(z, st, bng.reshape(1, 512), bnb.reshape(1, 512),
                              poolw.reshape(512, 1))
    coefb = jnp.repeat(poolc, 16)
    fpre = _sck_fitness_call(mp, _jacobi_recur_consts())(
        s2d.reshape(mp), dinv1d, src, dst, coefb, zeros1d[:mp])
    fit2d, key2d = _tanh_key_call(mp)(fpre.reshape(mp, 1))
    kk = _kk_call()(counts)
    keyrow = key2d.reshape(1, mp)
    batchrow = batch2d.reshape(1, mp)
    rank2d = _rank_call(mp)(key2d, batch2d, keyrow, batchrow)
    midx, mapping, bval, fval = _mapping_call(mp, mn)(rank2d, batch2d, fit2d,
                                                      kk)
    iota1d = jnp.arange(mp, dtype=I32)
    minv, bnew, fnew, xraw, srcn, dstn, degn = _sck_pool_call(mp, mn)(
        mapping.reshape(mp), midx.reshape(mp), bval.reshape(mp),
        fval.reshape(mp), iota1d, h, src, dst, zeros1d[:mn])
    fnew2d = fnew.reshape(mn, 1)
    bnew2d = bnew.reshape(mn, 1)
    ro_sum, ro_mx = _readout_call(mn)(xraw, fnew2d, bnew2d)
    return (xraw, fnew2d, bnew2d, srcn, dstn, degn, kk, ro_sum, ro_mx)


def kernel(x, edge_index, batch, W1, b1, bn1_g, bn1_b, pool1_w, pool1_c, W2,
           b2, bn2_g, bn2_b, pool2_w, pool2_c, W3, b3, bn3_g, bn3_b, pool3_w,
           pool3_c, lin1_W, lin1_b, bn4_g, bn4_b, lin2_W, lin2_b, bn5_g,
           bn5_b, lin3_W, lin3_b):
    mp0 = MP[0]
    x_p = jnp.pad(x, ((0, mp0 - NREAL), (0, 0)))
    batch2d = jnp.pad(batch.astype(I32), (0, mp0 - NREAL),
                      constant_values=NG).reshape(mp0, 1)
    epad = E2 - EREAL
    ep = jnp.arange(epad, dtype=I32)
    src0 = jnp.concatenate([edge_index[0].astype(I32), ep & 1023])
    dst0 = jnp.concatenate([edge_index[1].astype(I32),
                            (mp0 - 128) + (ep & 127)])
    zeros1d = jnp.zeros((mp0,), F32)
    zeros2d = jnp.zeros((mp0, 128), F32)

    deg0 = _sck_deg_call(mp0)(dst0, zeros1d)
    counts1 = _counts_call(mp0)(batch2d)

    (x2, f2, b2d, src1, dst1, deg1, kk1, s1m, x1mx) = _layer(
        0, x_p, None, batch2d, src0, dst0, deg0, W1, b1, bn1_g, bn1_b,
        pool1_w, pool1_c, counts1, zeros1d, zeros2d)
    (x3, f3, b3d, src2, dst2, deg2, kk2, s2m, x2mx) = _layer(
        1, x2, f2, b2d, src1, dst1, deg1, W2, b2, bn2_g, bn2_b, pool2_w,
        pool2_c, kk1, zeros1d, zeros2d)
    (_x4, _f4, _b4d, _s, _d, _dg, kk3, s3m, x3mx) = _layer(
        2, x3, f3, b3d, src2, dst2, deg2, W3, b3, bn3_g, bn3_b, pool3_w,
        pool3_c, kk2, zeros1d, zeros2d)

    kkc1 = kk1.reshape(NGP, 1).astype(F32)
    kkc2 = kk2.reshape(NGP, 1).astype(F32)
    kkc3 = kk3.reshape(NGP, 1).astype(F32)
    out = _head_call()(
        s1m, x1mx, kkc1, s2m, x2mx, kkc2, s3m, x3mx, kkc3,
        lin1_W[:512], lin1_W[512:], lin1_b.reshape(1, 512),
        bn4_g.reshape(1, 512), bn4_b.reshape(1, 512), lin2_W,
        lin2_b.reshape(1, 256), bn5_g.reshape(1, 256), bn5_b.reshape(1, 256),
        jnp.pad(lin3_W, ((0, 0), (0, 118))),
        jnp.pad(lin3_b, (0, 118)).reshape(1, 128))
    return out[:NG, :10]


# speedup vs baseline: 6.6629x; 1.1133x over previous
import functools
import math

import jax
import jax.numpy as jnp
from jax import lax
from jax.experimental import pallas as pl
from jax.experimental.pallas import tpu as pltpu
from jax.experimental.pallas import tpu_sc as plsc

NG = 20
NGP = 32
NREAL = 10000
EREAL = 160000
MP = (10240, 6144, 4096, 2048)
E2 = 161792
NT = 16
F32 = jnp.float32
I32 = jnp.int32


def _mm_pre_call(mp, kd, use_scale):
    r = 512

    def body(*refs):
        if use_scale:
            x_ref, w_ref, dinv_ref, rs_ref, hpre_ref, hq_ref = refs
            a = x_ref[...] * rs_ref[...]
        else:
            x_ref, w_ref, dinv_ref, hpre_ref, hq_ref = refs
            a = x_ref[...]
        acc = jnp.dot(a, w_ref[...], preferred_element_type=F32)
        hpre_ref[...] = acc
        scaled = acc * dinv_ref[...]
        for q in range(4):
            hq_ref[q] = scaled[:, q * 128:(q + 1) * 128]

    in_specs = [
        pl.BlockSpec((r, kd), lambda i: (i, 0)),
        pl.BlockSpec((kd, 512), lambda i: (0, 0)),
        pl.BlockSpec((r, 1), lambda i: (i, 0)),
    ]
    if use_scale:
        in_specs.append(pl.BlockSpec((r, 1), lambda i: (i, 0)))
    return pl.pallas_call(
        body,
        grid=(mp // r,),
        in_specs=in_specs,
        out_specs=[
            pl.BlockSpec((r, 512), lambda i: (i, 0)),
            pl.BlockSpec((4, r, 128), lambda i: (0, i, 0)),
        ],
        out_shape=[
            jax.ShapeDtypeStruct((mp, 512), F32),
            jax.ShapeDtypeStruct((4, mp, 128), F32),
        ],
    )


def _bnormA_call(mp):
    r = 512

    def body(s_ref, hpre_ref, dinv_ref, batch_ref, bias_ref, z_ref, st_ref):
        i = pl.program_id(0)
        mask = (batch_ref[...] < NG).astype(F32)
        dv = dinv_ref[...]
        sc = jnp.concatenate([s_ref[0], s_ref[1], s_ref[2], s_ref[3]], axis=1)
        zv = dv * sc + dv * dv * hpre_ref[...] + bias_ref[...]
        z_ref[...] = zv

        @pl.when(i == 0)
        def _():
            st_ref[...] = jnp.zeros_like(st_ref)

        st_ref[0:1, :] += jnp.sum(zv * mask, axis=0, keepdims=True)
        st_ref[1:2, :] += jnp.sum(zv * zv * mask, axis=0, keepdims=True)
        st_ref[2:3, :] += jnp.sum(mask) * jnp.ones((1, 512), F32)

    return pl.pallas_call(
        body,
        grid=(mp // r,),
        in_specs=[
            pl.BlockSpec((4, r, 128), lambda i: (0, i, 0)),
            pl.BlockSpec((r, 512), lambda i: (i, 0)),
            pl.BlockSpec((r, 1), lambda i: (i, 0)),
            pl.BlockSpec((r, 1), lambda i: (i, 0)),
            pl.BlockSpec((1, 512), lambda i: (0, 0)),
        ],
        out_specs=[
            pl.BlockSpec((r, 512), lambda i: (i, 0)),
            pl.BlockSpec((8, 512), lambda i: (0, 0)),
        ],
        out_shape=[
            jax.ShapeDtypeStruct((mp, 512), F32),
            jax.ShapeDtypeStruct((8, 512), F32),
        ],
    )


def _bnormB_call(mp):
    r = 512

    def body(z_ref, st_ref, g_ref, b_ref, w_ref, h_ref, s_ref):
        m = st_ref[2, 0]
        mu = st_ref[0:1, :] / m
        var = st_ref[1:2, :] / m - mu * mu
        hv = jnp.maximum(
            g_ref[...] * (z_ref[...] - mu) * lax.rsqrt(var + 1e-5) + b_ref[...],
            0.0)
        h_ref[...] = hv
        s_ref[...] = jnp.dot(hv, w_ref[...])

    return pl.pallas_call(
        body,
        grid=(mp // r,),
        in_specs=[
            pl.BlockSpec((r, 512), lambda i: (i, 0)),
            pl.BlockSpec((8, 512), lambda i: (0, 0)),
            pl.BlockSpec((1, 512), lambda i: (0, 0)),
            pl.BlockSpec((1, 512), lambda i: (0, 0)),
            pl.BlockSpec((512, 1), lambda i: (0, 0)),
        ],
        out_specs=[
            pl.BlockSpec((r, 512), lambda i: (i, 0)),
            pl.BlockSpec((r, 1), lambda i: (i, 0)),
        ],
        out_shape=[
            jax.ShapeDtypeStruct((mp, 512), F32),
            jax.ShapeDtypeStruct((mp, 1), F32),
        ],
    )


def _tanh_key_call(mp):
    r = 512

    def body(f_ref, fit_ref, key_ref):
        fit = jnp.tanh(f_ref[...])
        fit_ref[...] = fit
        u = lax.bitcast_convert_type(fit, I32)
        key_ref[...] = jnp.where(u >= 0, u, u ^ 0x7FFFFFFF)

    return pl.pallas_call(
        body,
        grid=(mp // r,),
        in_specs=[pl.BlockSpec((r, 1), lambda i: (i, 0))],
        out_specs=[pl.BlockSpec((r, 1), lambda i: (i, 0)),
                   pl.BlockSpec((r, 1), lambda i: (i, 0))],
        out_shape=[jax.ShapeDtypeStruct((mp, 1), F32),
                   jax.ShapeDtypeStruct((mp, 1), I32)],
    )


def _dinv_call(mp):
    def body(deg_ref, d2_ref, d1_ref):
        dv = lax.rsqrt(jnp.clip(deg_ref[...] + 1.0, 1.0))
        d2_ref[...] = dv
        d1_ref[...] = dv

    return pl.pallas_call(
        body,
        grid=(1,),
        in_specs=[pl.BlockSpec((mp, 1), lambda i: (0, 0))],
        out_specs=[pl.BlockSpec((mp, 1), lambda i: (0, 0)),
                   pl.BlockSpec((mp, 1), lambda i: (0, 0))],
        out_shape=[jax.ShapeDtypeStruct((mp, 1), F32),
                   jax.ShapeDtypeStruct((mp, 1), F32)],
    )


def _counts_call(mp):
    r = 512

    def body(b_ref, c_ref):
        i = pl.program_id(0)

        @pl.when(i == 0)
        def _():
            c_ref[...] = jnp.zeros_like(c_ref)

        gi = lax.broadcasted_iota(I32, (1, NGP), 1)
        oh = (b_ref[...] == gi).astype(I32)
        c_ref[...] += jnp.sum(oh, axis=0, keepdims=True)

    return pl.pallas_call(
        body,
        grid=(mp // r,),
        in_specs=[pl.BlockSpec((r, 1), lambda i: (i, 0))],
        out_specs=pl.BlockSpec((1, NGP), lambda i: (0, 0)),
        out_shape=jax.ShapeDtypeStruct((1, NGP), I32),
    )


def _kk_call():
    def body(c_ref, k_ref):
        c = c_ref[...]
        gi = lax.broadcasted_iota(I32, (1, NGP), 1)
        k_ref[...] = jnp.where((c > 0) & (gi < NG),
                               jnp.maximum(1, (c + 1) // 2), 0)

    return pl.pallas_call(
        body,
        grid=(1,),
        in_specs=[pl.BlockSpec((1, NGP), lambda i: (0, 0))],
        out_specs=pl.BlockSpec((1, NGP), lambda i: (0, 0)),
        out_shape=jax.ShapeDtypeStruct((1, NGP), I32),
    )


def _rank_call(mp):
    ri, rj = 256, 512

    def body(kc_ref, bc_ref, kr_ref, br_ref, o_ref):
        i = pl.program_id(0)
        j = pl.program_id(1)

        @pl.when(j == 0)
        def _():
            o_ref[...] = jnp.zeros_like(o_ref)

        ii = i * ri + lax.broadcasted_iota(I32, (ri, 1), 0)
        jj = j * rj + lax.broadcasted_iota(I32, (1, rj), 1)
        kc = kc_ref[...]
        kr = kr_ref[...]
        same = bc_ref[...] == br_ref[...]
        cmp = (kr > kc) | ((kr == kc) & (jj < ii))
        o_ref[...] += jnp.sum((same & cmp).astype(I32), axis=1, keepdims=True)

    return pl.pallas_call(
        body,
        grid=(mp // ri, mp // rj),
        in_specs=[
            pl.BlockSpec((ri, 1), lambda i, j: (i, 0)),
            pl.BlockSpec((ri, 1), lambda i, j: (i, 0)),
            pl.BlockSpec((1, rj), lambda i, j: (0, j)),
            pl.BlockSpec((1, rj), lambda i, j: (0, j)),
        ],
        out_specs=pl.BlockSpec((ri, 1), lambda i, j: (i, 0)),
        out_shape=jax.ShapeDtypeStruct((mp, 1), I32),
    )


def _mapping_call(mp, mn):
    r = 512
    trash0 = mn - 128

    def body(rank_ref, b_ref, fit_ref, kk_ref, midx_ref, map_ref, bv_ref,
             fv_ref):
        i = pl.program_id(0)
        gi = lax.broadcasted_iota(I32, (1, NGP), 1)
        kkv = kk_ref[...]
        b = b_ref[...]
        rank = rank_ref[...]
        kk_i = jnp.sum(jnp.where(b == gi, kkv, 0), axis=1, keepdims=True)
        off_i = jnp.sum(jnp.where(gi < b, kkv, 0), axis=1, keepdims=True)
        keep = (rank < kk_i) & (b < NG)
        mapping = jnp.where(keep, off_i + rank, -1)
        map_ref[...] = mapping
        rowid = i * r + lax.broadcasted_iota(I32, (r, 1), 0)
        midx_ref[...] = jnp.where(keep, mapping, trash0 + (rowid & 127))
        bv_ref[...] = jnp.where(keep, b, NG)
        fv_ref[...] = jnp.where(keep, fit_ref[...], 0.0)

    return pl.pallas_call(
        body,
        grid=(mp // r,),
        in_specs=[
            pl.BlockSpec((r, 1), lambda i: (i, 0)),
            pl.BlockSpec((r, 1), lambda i: (i, 0)),
            pl.BlockSpec((r, 1), lambda i: (i, 0)),
            pl.BlockSpec((1, NGP), lambda i: (0, 0)),
        ],
        out_specs=[pl.BlockSpec((r, 1), lambda i: (i, 0))] * 4,
        out_shape=[jax.ShapeDtypeStruct((mp, 1), I32),
                   jax.ShapeDtypeStruct((mp, 1), I32),
                   jax.ShapeDtypeStruct((mp, 1), I32),
                   jax.ShapeDtypeStruct((mp, 1), F32)],
    )


def _readout_call(mp):
    r = 512

    def body(x_ref, f_ref, b_ref, sum_ref, mx_ref):
        i = pl.program_id(0)
        xv = x_ref[...] * f_ref[...]
        b = b_ref[...]
        gi = lax.broadcasted_iota(I32, (1, NGP), 1)
        oh = (b == gi).astype(F32)

        @pl.when(i == 0)
        def _():
            sum_ref[...] = jnp.zeros_like(sum_ref)
            mx_ref[...] = jnp.full_like(mx_ref, -jnp.inf)

        sum_ref[...] += lax.dot_general(oh, xv, (((0,), (0,)), ((), ())),
                                        preferred_element_type=F32)
        for g in range(NG):
            row = jnp.max(jnp.where(b == g, xv, -jnp.inf), axis=0,
                          keepdims=True)
            mx_ref[g:g + 1, :] = jnp.maximum(mx_ref[g:g + 1, :], row)

    return pl.pallas_call(
        body,
        grid=(mp // r,),
        in_specs=[
            pl.BlockSpec((r, 512), lambda i: (i, 0)),
            pl.BlockSpec((r, 1), lambda i: (i, 0)),
            pl.BlockSpec((r, 1), lambda i: (i, 0)),
        ],
        out_specs=[pl.BlockSpec((NGP, 512), lambda i: (0, 0)),
                   pl.BlockSpec((NGP, 512), lambda i: (0, 0))],
        out_shape=[jax.ShapeDtypeStruct((NGP, 512), F32),
                   jax.ShapeDtypeStruct((NGP, 512), F32)],
    )


def _head_call():
    def bn(t, g, b, rm):
        mu = jnp.sum(t * rm, axis=0, keepdims=True) / float(NG)
        var = jnp.sum(rm * (t - mu) ** 2, axis=0, keepdims=True) / float(NG)
        return jnp.maximum(g * (t - mu) * lax.rsqrt(var + 1e-5) + b, 0.0)

    def body(s1, m1, k1, s2, m2, k2, s3, m3, k3, wa, wb, l1b, g4, b4, w2, l2b,
             g5, b5, w3, l3b, o_ref):
        rm = (lax.broadcasted_iota(I32, (NGP, 1), 0) < NG).astype(F32)

        def xl(s_ref, m_ref, k_ref):
            kc = k_ref[...].astype(F32)
            mean = s_ref[...] / jnp.clip(kc, 1.0)
            mx = jnp.where(kc > 0, m_ref[...], 0.0)
            return mx, mean

        x1m, x1a = xl(s1, m1, k1)
        x2m, x2a = xl(s2, m2, k2)
        x3m, x3a = xl(s3, m3, k3)
        mxz = (x1m + x2m + x3m) / 3.0
        mnz = (x1a + x2a + x3a) / 3.0
        t = (jnp.dot(mxz, wa[...], preferred_element_type=F32)
             + jnp.dot(mnz, wb[...], preferred_element_type=F32) + l1b[...])
        t = bn(t, g4[...], b4[...], rm)
        u = jnp.dot(t, w2[...], preferred_element_type=F32) + l2b[...]
        u = bn(u, g5[...], b5[...], rm)
        logits = jnp.dot(u, w3[...], preferred_element_type=F32) + l3b[...]
        lanem = lax.broadcasted_iota(I32, (1, 128), 1) < 10
        mm = jnp.max(jnp.where(lanem, logits, -jnp.inf), axis=1, keepdims=True)
        lse = jnp.log(jnp.sum(jnp.where(lanem, jnp.exp(logits - mm), 0.0),
                              axis=1, keepdims=True)) + mm
        o_ref[...] = logits - lse

    full = lambda shape: pl.BlockSpec(shape, lambda: tuple(0 for _ in shape))
    in_specs = []
    for _ in range(3):
        in_specs += [full((NGP, 512)), full((NGP, 512)), full((NGP, 1))]
    in_specs += [full((512, 512)), full((512, 512)), full((1, 512)),
                 full((1, 512)), full((1, 512)), full((512, 256)),
                 full((1, 256)), full((1, 256)), full((1, 256)),
                 full((256, 128)), full((1, 128))]
    return pl.pallas_call(
        body,
        grid=(),
        in_specs=in_specs,
        out_specs=full((NGP, 128)),
        out_shape=jax.ShapeDtypeStruct((NGP, 128), F32),
    )


def _vfill(ref, n, val, dtype):
    for k in range(n // 16):
        ref[pl.ds(k * 16, 16)] = jnp.full((16,), val, dtype)


_MESH = None


def _mesh():
    global _MESH
    if _MESH is None:
        _MESH = plsc.VectorSubcoreMesh(core_axis_name="c", subcore_axis_name="s")
    return _MESH


def _sck_deg_call(mp):
    sl = mp // NT
    ept = E2 // NT
    nb = ept // 128

    @functools.partial(
        pl.kernel,
        out_type=jax.ShapeDtypeStruct((mp,), F32),
        mesh=_mesh(),
        scratch_types=[
            pltpu.VMEM((128,), I32),
            pltpu.VMEM((128,), F32),
            pltpu.VMEM_SHARED((mp,), F32),
        ],
    )
    def k(dst_hbm, zeros_hbm, out_hbm, idxv, onesv, acc):
        c = lax.axis_index("c")
        s = lax.axis_index("s")

        @pl.when(c == 0)
        def _():
            pltpu.sync_copy(zeros_hbm.at[pl.ds(s * sl, sl)],
                            acc.at[pl.ds(s * sl, sl)])
            _vfill(onesv, 128, 1.0, F32)
            plsc.subcore_barrier()

            def blk(b, carry):
                base = s * ept + b * 128
                pltpu.sync_copy(dst_hbm.at[pl.ds(base, 128)], idxv)
                pltpu.sync_copy(onesv, acc.at[idxv], add=True)
                return carry

            lax.fori_loop(0, nb, blk, 0)
            plsc.subcore_barrier()
            pltpu.sync_copy(acc.at[pl.ds(s * sl, sl)],
                            out_hbm.at[pl.ds(s * sl, sl)])

    return k


def _sck_conv_call(mp):
    sl = mp // NT
    ept = E2 // NT
    nb = ept // 128

    @functools.partial(
        pl.kernel,
        out_type=jax.ShapeDtypeStruct((4, mp, 128), F32),
        mesh=_mesh(),
        scratch_types=[
            pltpu.VMEM((128,), I32),
            pltpu.VMEM((128,), I32),
            pltpu.VMEM((128,), I32),
            pltpu.VMEM((128,), I32),
            pltpu.VMEM((128, 128), F32),
            pltpu.VMEM((128, 128), F32),
            pltpu.VMEM_SHARED((mp, 128), F32),
            pltpu.SemaphoreType.DMA,
            pltpu.SemaphoreType.DMA,
        ],
    )
    def k(hq_hbm, src_hbm, dst_hbm, zeros_hbm, out_hbm, sidx0, didx0, sidx1,
          didx1, rows0, rows1, acc, sem0, sem1):
        c = lax.axis_index("c")
        s = lax.axis_index("s")

        def one_chunk(q):
            pltpu.sync_copy(zeros_hbm.at[pl.ds(s * sl, sl)],
                            acc.at[pl.ds(s * sl, sl)])
            plsc.subcore_barrier()

            def start(b, sidx, didx, rows, sem):
                base = s * ept + b * 128
                pltpu.sync_copy(src_hbm.at[pl.ds(base, 128)], sidx)
                pltpu.sync_copy(dst_hbm.at[pl.ds(base, 128)], didx)
                pltpu.async_copy(hq_hbm.at[q].at[sidx], rows, sem)

            def drain_scatter(sidx, didx, rows, sem):
                pltpu.make_async_copy(hq_hbm.at[q].at[sidx], rows, sem).wait()
                pltpu.sync_copy(rows, acc.at[didx], add=True)

            start(0, sidx0, didx0, rows0, sem0)

            def pair(p, carry):
                b0 = 2 * p
                start(b0 + 1, sidx1, didx1, rows1, sem1)
                drain_scatter(sidx0, didx0, rows0, sem0)

                @pl.when(b0 + 2 < nb)
                def _():
                    start(b0 + 2, sidx0, didx0, rows0, sem0)

                drain_scatter(sidx1, didx1, rows1, sem1)
                return carry

            lax.fori_loop(0, nb // 2, pair, 0)
            if nb % 2:
                drain_scatter(sidx0, didx0, rows0, sem0)
            plsc.subcore_barrier()
            pltpu.sync_copy(acc.at[pl.ds(s * sl, sl)],
                            out_hbm.at[q].at[pl.ds(s * sl, sl)])
            plsc.subcore_barrier()

        for ci in range(2):
            @pl.when(c == ci)
            def _():
                one_chunk(2 * ci)
                one_chunk(2 * ci + 1)

    return k


def _sck_fitness_call(mp, acoefs):
    sl = mp // NT
    ept = E2 // NT
    nb = ept // 128

    @functools.partial(
        pl.kernel,
        out_type=jax.ShapeDtypeStruct((mp,), F32),
        mesh=_mesh(),
        scratch_types=[
            pltpu.VMEM((128,), I32),
            pltpu.VMEM((128,), I32),
            pltpu.VMEM((128,), F32),
            pltpu.VMEM((sl,), F32),
            pltpu.VMEM((sl,), F32),
            pltpu.VMEM((sl,), F32),
            pltpu.VMEM((sl,), F32),
            pltpu.VMEM((sl,), F32),
            pltpu.VMEM((sl,), F32),
            pltpu.VMEM((64,), F32),
            pltpu.VMEM_SHARED((mp,), F32),
            pltpu.VMEM_SHARED((mp,), F32),
            pltpu.SemaphoreType.DMA,
        ],
    )
    def k(s_hbm, dinv_hbm, src_hbm, dst_hbm, coefb_hbm, zeros_hbm, out_hbm,
          sidx, didx, vals, dsl, stage, pm2, pm1, pk, osl, cvec, upub, acc,
          sem):
        c = lax.axis_index("c")
        s = lax.axis_index("s")

        @pl.when(c == 0)
        def _():
            base = s * sl
            pltpu.sync_copy(dinv_hbm.at[pl.ds(base, sl)], dsl)
            pltpu.sync_copy(s_hbm.at[pl.ds(base, sl)], pm2)
            pltpu.sync_copy(coefb_hbm, cvec)

            def coef(kc):
                return cvec[pl.ds(kc * 16, 16)]

            def spmv(v_ref):
                for t in range(sl // 16):
                    d16 = pl.ds(t * 16, 16)
                    stage[d16] = dsl[d16] * v_ref[d16]
                pltpu.sync_copy(stage, upub.at[pl.ds(base, sl)])
                pltpu.sync_copy(zeros_hbm.at[pl.ds(base, sl)],
                                acc.at[pl.ds(base, sl)])
                plsc.subcore_barrier()

                def blk(b, carry):
                    eb = s * ept + b * 128
                    pltpu.sync_copy(src_hbm.at[pl.ds(eb, 128)], sidx)
                    pltpu.sync_copy(dst_hbm.at[pl.ds(eb, 128)], didx)
                    pltpu.async_copy(upub.at[sidx], vals, sem).wait()
                    pltpu.sync_copy(vals, acc.at[didx], add=True)
                    return carry

                lax.fori_loop(0, nb, blk, 0)
                plsc.subcore_barrier()
                pltpu.sync_copy(acc.at[pl.ds(base, sl)], stage)

            c0 = coef(0)
            for t in range(sl // 16):
                d16 = pl.ds(t * 16, 16)
                osl[d16] = c0 * pm2[d16]
            spmv(pm2)
            c1c = coef(1)
            for t in range(sl // 16):
                d16 = pl.ds(t * 16, 16)
                av = dsl[d16] * stage[d16] + dsl[d16] * dsl[d16] * pm2[d16]
                pm1[d16] = 2.0 * av
                osl[d16] = osl[d16] + c1c * pm1[d16]
            for kk_, (cc1, cc2, cc3, cc4) in enumerate(acoefs):
                spmv(pm1)
                ck = coef(kk_ + 2)
                for t in range(sl // 16):
                    d16 = pl.ds(t * 16, 16)
                    av = dsl[d16] * stage[d16] + dsl[d16] * dsl[d16] * pm1[d16]
                    pkv = (cc2 * pm1[d16] + cc3 * av - cc4 * pm2[d16]) / cc1
                    pk[d16] = pkv
                    osl[d16] = osl[d16] + ck * pkv
                for t in range(sl // 16):
                    d16 = pl.ds(t * 16, 16)
                    pm2[d16] = pm1[d16]
                    pm1[d16] = pk[d16]
            pltpu.sync_copy(osl, out_hbm.at[pl.ds(base, sl)])

    return k


def _sck_pool_call(mp, mn, phases=(0, 1, 2, 3)):
    sl = mp // NT
    sln = mn // NT
    ept = E2 // NT
    nb = ept // 128
    trash0 = mn - 128

    out_type = [
        jax.ShapeDtypeStruct((mn,), I32),
        jax.ShapeDtypeStruct((mn,), I32),
        jax.ShapeDtypeStruct((mn,), F32),
        jax.ShapeDtypeStruct((mn, 512), F32),
        jax.ShapeDtypeStruct((E2,), I32),
        jax.ShapeDtypeStruct((E2,), I32),
        jax.ShapeDtypeStruct((mn,), F32),
    ]

    @functools.partial(
        pl.kernel,
        out_type=out_type,
        mesh=_mesh(),
        scratch_types=[
            pltpu.VMEM((128,), I32),
            pltpu.VMEM((128,), I32),
            pltpu.VMEM((128,), I32),
            pltpu.VMEM((128,), F32),
            pltpu.VMEM((128,), F32),
            pltpu.VMEM((128,), I32),
            pltpu.VMEM((128,), F32),
            pltpu.VMEM((64,), I32),
            pltpu.VMEM((64, 512), F32),
            pltpu.VMEM((128,), I32),
            pltpu.VMEM((128,), I32),
            pltpu.VMEM_SHARED((mn,), F32),
            pltpu.SemaphoreType.DMA,
        ],
    )
    def k(map_hbm, midx_hbm, bval_hbm, fval_hbm, iota_hbm, h_hbm, src_hbm,
          dst_hbm, zeros_hbm, minv_hbm, bnew_hbm, fnew_hbm, xraw_hbm,
          srcn_hbm, dstn_hbm, deg_hbm, ia, ib, iv, fv, ones, dbi, dbf, ri,
          rows, smv, dmv, dacc, sem):
        c = lax.axis_index("c")
        s = lax.axis_index("s")

        @pl.when(c == 0)
        def _():
            if 0 not in phases:
                return
            _vfill(dbi, 128, NG, I32)
            _vfill(dbf, 128, 0.0, F32)
            for bidx in range(sln // 128):
                b0 = s * sln + bidx * 128
                pltpu.sync_copy(dbf, fnew_hbm.at[pl.ds(b0, 128)])
                pltpu.sync_copy(dbi, bnew_hbm.at[pl.ds(b0, 128)])
            _vfill(dbi, 128, 0, I32)
            for bidx in range(sln // 128):
                b0 = s * sln + bidx * 128
                pltpu.sync_copy(dbi, minv_hbm.at[pl.ds(b0, 128)])
            plsc.subcore_barrier()

            if 1 not in phases:
                return
            nbk = mp // 128

            def scat(b0):
                pltpu.sync_copy(midx_hbm.at[pl.ds(b0, 128)], ia)
                pltpu.sync_copy(iota_hbm.at[pl.ds(b0, 128)], iv)
                pltpu.sync_copy(iv, minv_hbm.at[ia])
                pltpu.sync_copy(bval_hbm.at[pl.ds(b0, 128)], iv)
                pltpu.sync_copy(iv, bnew_hbm.at[ia])
                pltpu.sync_copy(fval_hbm.at[pl.ds(b0, 128)], fv)
                pltpu.sync_copy(fv, fnew_hbm.at[ia])

            for j in range((nbk + NT - 1) // NT):
                bi = s + j * NT
                if (j + 1) * NT <= nbk:
                    scat(bi * 128)
                else:
                    @pl.when(bi < nbk)
                    def _():
                        scat(bi * 128)
            plsc.subcore_barrier()

            if 2 not in phases:
                return

            def gat(r0):
                pltpu.sync_copy(minv_hbm.at[pl.ds(r0, 64)], ri)
                pltpu.async_copy(h_hbm.at[ri], rows, sem).wait()
                pltpu.sync_copy(rows, xraw_hbm.at[pl.ds(r0, 64)])

            for bidx in range(sln // 64):
                gat(s * sln + bidx * 64)

        @pl.when(c == 1)
        def _():
            if 3 not in phases:
                return
            _vfill(ones, 128, 1.0, F32)
            pltpu.sync_copy(zeros_hbm.at[pl.ds(s * sln, sln)],
                            dacc.at[pl.ds(s * sln, sln)])
            plsc.subcore_barrier()
            io16 = lax.iota(I32, 16)

            def blk(b, carry):
                eb = s * ept + b * 128
                pltpu.sync_copy(src_hbm.at[pl.ds(eb, 128)], ia)
                pltpu.sync_copy(dst_hbm.at[pl.ds(eb, 128)], ib)
                pltpu.async_copy(map_hbm.at[ia], smv, sem).wait()
                pltpu.async_copy(map_hbm.at[ib], dmv, sem).wait()
                for t in range(8):
                    d16 = pl.ds(t * 16, 16)
                    sm = smv[d16]
                    dm = dmv[d16]
                    keep = (sm >= 0) & (dm >= 0)
                    eidx = eb + t * 16 + io16
                    ia[d16] = jnp.where(keep, sm, eidx & 1023)
                    ib[d16] = jnp.where(keep, dm, trash0 + (eidx & 127))
                pltpu.sync_copy(ia, srcn_hbm.at[pl.ds(eb, 128)])
                pltpu.sync_copy(ib, dstn_hbm.at[pl.ds(eb, 128)])
                pltpu.sync_copy(ones, dacc.at[ib], add=True)
                return carry

            lax.fori_loop(0, nb, blk, 0)
            plsc.subcore_barrier()
            pltpu.sync_copy(dacc.at[pl.ds(s * sln, sln)],
                            deg_hbm.at[pl.ds(s * sln, sln)])

    return k


def _jacobi_recur_consts():
    a, b = 1.0, 1.0
    out = []
    for k in range(2, 4):
        c1 = 2.0 * k * (k + a + b) * (2 * k + a + b - 2)
        c2 = (2 * k + a + b - 1) * (a * a - b * b)
        c3 = (2 * k + a + b - 2) * (2 * k + a + b - 1) * (2 * k + a + b)
        c4 = 2.0 * (k + a - 1) * (k + b - 1) * (2 * k + a + b)
        out.append((c1, c2, c3, c4))
    return tuple(out)


def _layer(li, x, rowscale, batch2d, src, dst, deg1d, W, bias, bng, bnb,
           poolw, poolc, counts, zeros1d, zeros2d):
    mp, mn = MP[li], MP[li + 1]
    deg2d = deg1d.reshape(mp, 1)
    dinv2d, dinv1d2 = _dinv_call(mp)(deg2d)
    dinv1d = dinv1d2.reshape(mp)
    if rowscale is None:
        hpre, hq = _mm_pre_call(mp, W.shape[0], False)(x, W, dinv2d)
    else:
        hpre, hq = _mm_pre_call(mp, W.shape[0], True)(x, W, dinv2d, rowscale)
    S = _sck_conv_call(mp)(hq, src, dst, zeros2d[:mp])
    z, st = _bnormA_call(mp)(S, hpre, dinv2d, batch2d, bias.reshape(1, 512))
    h, s2d = _bnormB_call(mp)(z, st, bng.reshape(1, 512), bnb.reshape(1, 512),
                              poolw.reshape(512, 1))
    coefb = jnp.repeat(poolc, 16)
    fpre = _sck_fitness_call(mp, _jacobi_recur_consts())(
        s2d.reshape(mp), dinv1d, src, dst, coefb, zeros1d[:mp])
    fit2d, key2d = _tanh_key_call(mp)(fpre.reshape(mp, 1))
    kk = _kk_call()(counts)
    keyrow = key2d.reshape(1, mp)
    batchrow = batch2d.reshape(1, mp)
    rank2d = _rank_call(mp)(key2d, batch2d, keyrow, batchrow)
    midx, mapping, bval, fval = _mapping_call(mp, mn)(rank2d, batch2d, fit2d,
                                                      kk)
    iota1d = jnp.arange(mp, dtype=I32)
    minv, bnew, fnew, xraw, srcn, dstn, degn = _sck_pool_call(mp, mn)(
        mapping.reshape(mp), midx.reshape(mp), bval.reshape(mp),
        fval.reshape(mp), iota1d, h, src, dst, zeros1d[:mn])
    fnew2d = fnew.reshape(mn, 1)
    bnew2d = bnew.reshape(mn, 1)
    ro_sum, ro_mx = _readout_call(mn)(xraw, fnew2d, bnew2d)
    return (xraw, fnew2d, bnew2d, srcn, dstn, degn, kk, ro_sum, ro_mx)


def kernel(x, edge_index, batch, W1, b1, bn1_g, bn1_b, pool1_w, pool1_c, W2,
           b2, bn2_g, bn2_b, pool2_w, pool2_c, W3, b3, bn3_g, bn3_b, pool3_w,
           pool3_c, lin1_W, lin1_b, bn4_g, bn4_b, lin2_W, lin2_b, bn5_g,
           bn5_b, lin3_W, lin3_b):
    mp0 = MP[0]
    x_p = jnp.pad(x, ((0, mp0 - NREAL), (0, 0)))
    batch2d = jnp.pad(batch.astype(I32), (0, mp0 - NREAL),
                      constant_values=NG).reshape(mp0, 1)
    epad = E2 - EREAL
    ep = jnp.arange(epad, dtype=I32)
    src0 = jnp.concatenate([edge_index[0].astype(I32), ep & 1023])
    dst0 = jnp.concatenate([edge_index[1].astype(I32),
                            (mp0 - 128) + (ep & 127)])
    zeros1d = jnp.zeros((mp0,), F32)
    zeros2d = jnp.zeros((mp0, 128), F32)

    deg0 = _sck_deg_call(mp0)(dst0, zeros1d)
    counts1 = _counts_call(mp0)(batch2d)

    (x2, f2, b2d, src1, dst1, deg1, kk1, s1m, x1mx) = _layer(
        0, x_p, None, batch2d, src0, dst0, deg0, W1, b1, bn1_g, bn1_b,
        pool1_w, pool1_c, counts1, zeros1d, zeros2d)
    (x3, f3, b3d, src2, dst2, deg2, kk2, s2m, x2mx) = _layer(
        1, x2, f2, b2d, src1, dst1, deg1, W2, b2, bn2_g, bn2_b, pool2_w,
        pool2_c, kk1, zeros1d, zeros2d)
    (_x4, _f4, _b4d, _s, _d, _dg, kk3, s3m, x3mx) = _layer(
        2, x3, f3, b3d, src2, dst2, deg2, W3, b3, bn3_g, bn3_b, pool3_w,
        pool3_c, kk2, zeros1d, zeros2d)

    kkc1 = kk1.reshape(NGP, 1).astype(F32)
    kkc2 = kk2.reshape(NGP, 1).astype(F32)
    kkc3 = kk3.reshape(NGP, 1).astype(F32)
    out = _head_call()(
        s1m, x1mx, kkc1, s2m, x2mx, kkc2, s3m, x3mx, kkc3,
        lin1_W[:512], lin1_W[512:], lin1_b.reshape(1, 512),
        bn4_g.reshape(1, 512), bn4_b.reshape(1, 512), lin2_W,
        lin2_b.reshape(1, 256), bn5_g.reshape(1, 256), bn5_b.reshape(1, 256),
        jnp.pad(lin3_W, ((0, 0), (0, 118))),
        jnp.pad(lin3_b, (0, 118)).reshape(1, 128))
    return out[:NG, :10]


# speedup vs baseline: 6.7979x; 1.0203x over previous
import functools
import math

import jax
import jax.numpy as jnp
from jax import lax
from jax.experimental import pallas as pl
from jax.experimental.pallas import tpu as pltpu
from jax.experimental.pallas import tpu_sc as plsc

NG = 20
NGP = 32
NREAL = 10000
EREAL = 160000
MP = (10240, 6144, 4096, 2048)
E2 = 161792
NT = 16
F32 = jnp.float32
I32 = jnp.int32


def _mm_pre_call(mp, kd, use_scale):
    r = 512

    def body(*refs):
        if use_scale:
            x_ref, w_ref, dinv_ref, rs_ref, hpre_ref, hq_ref = refs
            a = x_ref[...] * rs_ref[...]
        else:
            x_ref, w_ref, dinv_ref, hpre_ref, hq_ref = refs
            a = x_ref[...]
        acc = jnp.dot(a, w_ref[...], preferred_element_type=F32)
        hpre_ref[...] = acc
        scaled = acc * dinv_ref[...]
        for q in range(4):
            hq_ref[q] = scaled[:, q * 128:(q + 1) * 128]

    in_specs = [
        pl.BlockSpec((r, kd), lambda i: (i, 0)),
        pl.BlockSpec((kd, 512), lambda i: (0, 0)),
        pl.BlockSpec((r, 1), lambda i: (i, 0)),
    ]
    if use_scale:
        in_specs.append(pl.BlockSpec((r, 1), lambda i: (i, 0)))
    return pl.pallas_call(
        body,
        grid=(mp // r,),
        in_specs=in_specs,
        out_specs=[
            pl.BlockSpec((r, 512), lambda i: (i, 0)),
            pl.BlockSpec((4, r, 128), lambda i: (0, i, 0)),
        ],
        out_shape=[
            jax.ShapeDtypeStruct((mp, 512), F32),
            jax.ShapeDtypeStruct((4, mp, 128), F32),
        ],
    )


def _bnormA_call(mp):
    r = 512

    def body(s_ref, hpre_ref, dinv_ref, batch_ref, bias_ref, z_ref, st_ref):
        i = pl.program_id(0)
        mask = (batch_ref[...] < NG).astype(F32)
        dv = dinv_ref[...]
        sc = jnp.concatenate([s_ref[0], s_ref[1], s_ref[2], s_ref[3]], axis=1)
        zv = dv * sc + dv * dv * hpre_ref[...] + bias_ref[...]
        z_ref[...] = zv

        @pl.when(i == 0)
        def _():
            st_ref[...] = jnp.zeros_like(st_ref)

        st_ref[0:1, :] += jnp.sum(zv * mask, axis=0, keepdims=True)
        st_ref[1:2, :] += jnp.sum(zv * zv * mask, axis=0, keepdims=True)
        st_ref[2:3, :] += jnp.sum(mask) * jnp.ones((1, 512), F32)

    return pl.pallas_call(
        body,
        grid=(mp // r,),
        in_specs=[
            pl.BlockSpec((4, r, 128), lambda i: (0, i, 0)),
            pl.BlockSpec((r, 512), lambda i: (i, 0)),
            pl.BlockSpec((r, 1), lambda i: (i, 0)),
            pl.BlockSpec((r, 1), lambda i: (i, 0)),
            pl.BlockSpec((1, 512), lambda i: (0, 0)),
        ],
        out_specs=[
            pl.BlockSpec((r, 512), lambda i: (i, 0)),
            pl.BlockSpec((8, 512), lambda i: (0, 0)),
        ],
        out_shape=[
            jax.ShapeDtypeStruct((mp, 512), F32),
            jax.ShapeDtypeStruct((8, 512), F32),
        ],
    )


def _bnormB_call(mp):
    r = 512

    def body(z_ref, st_ref, g_ref, b_ref, w_ref, h_ref, s_ref):
        m = st_ref[2, 0]
        mu = st_ref[0:1, :] / m
        var = st_ref[1:2, :] / m - mu * mu
        hv = jnp.maximum(
            g_ref[...] * (z_ref[...] - mu) * lax.rsqrt(var + 1e-5) + b_ref[...],
            0.0)
        h_ref[...] = hv
        s_ref[...] = jnp.dot(hv, w_ref[...])

    return pl.pallas_call(
        body,
        grid=(mp // r,),
        in_specs=[
            pl.BlockSpec((r, 512), lambda i: (i, 0)),
            pl.BlockSpec((8, 512), lambda i: (0, 0)),
            pl.BlockSpec((1, 512), lambda i: (0, 0)),
            pl.BlockSpec((1, 512), lambda i: (0, 0)),
            pl.BlockSpec((512, 1), lambda i: (0, 0)),
        ],
        out_specs=[
            pl.BlockSpec((r, 512), lambda i: (i, 0)),
            pl.BlockSpec((r, 1), lambda i: (i, 0)),
        ],
        out_shape=[
            jax.ShapeDtypeStruct((mp, 512), F32),
            jax.ShapeDtypeStruct((mp, 1), F32),
        ],
    )


def _tanh_key_call(mp):
    r = 512

    def body(f_ref, fit_ref, key_ref):
        fit = jnp.tanh(f_ref[...])
        fit_ref[...] = fit
        u = lax.bitcast_convert_type(fit, I32)
        key_ref[...] = jnp.where(u >= 0, u, u ^ 0x7FFFFFFF)

    return pl.pallas_call(
        body,
        grid=(mp // r,),
        in_specs=[pl.BlockSpec((r, 1), lambda i: (i, 0))],
        out_specs=[pl.BlockSpec((r, 1), lambda i: (i, 0)),
                   pl.BlockSpec((r, 1), lambda i: (i, 0))],
        out_shape=[jax.ShapeDtypeStruct((mp, 1), F32),
                   jax.ShapeDtypeStruct((mp, 1), I32)],
    )


def _dinv_call(mp):
    def body(deg_ref, d2_ref, d1_ref):
        dv = lax.rsqrt(jnp.clip(deg_ref[...] + 1.0, 1.0))
        d2_ref[...] = dv
        d1_ref[...] = dv

    return pl.pallas_call(
        body,
        grid=(1,),
        in_specs=[pl.BlockSpec((mp, 1), lambda i: (0, 0))],
        out_specs=[pl.BlockSpec((mp, 1), lambda i: (0, 0)),
                   pl.BlockSpec((mp, 1), lambda i: (0, 0))],
        out_shape=[jax.ShapeDtypeStruct((mp, 1), F32),
                   jax.ShapeDtypeStruct((mp, 1), F32)],
    )


def _counts_call(mp):
    r = 512

    def body(b_ref, c_ref):
        i = pl.program_id(0)

        @pl.when(i == 0)
        def _():
            c_ref[...] = jnp.zeros_like(c_ref)

        gi = lax.broadcasted_iota(I32, (1, NGP), 1)
        oh = (b_ref[...] == gi).astype(I32)
        c_ref[...] += jnp.sum(oh, axis=0, keepdims=True)

    return pl.pallas_call(
        body,
        grid=(mp // r,),
        in_specs=[pl.BlockSpec((r, 1), lambda i: (i, 0))],
        out_specs=pl.BlockSpec((1, NGP), lambda i: (0, 0)),
        out_shape=jax.ShapeDtypeStruct((1, NGP), I32),
    )


def _kk_call():
    def body(c_ref, k_ref):
        c = c_ref[...]
        gi = lax.broadcasted_iota(I32, (1, NGP), 1)
        k_ref[...] = jnp.where((c > 0) & (gi < NG),
                               jnp.maximum(1, (c + 1) // 2), 0)

    return pl.pallas_call(
        body,
        grid=(1,),
        in_specs=[pl.BlockSpec((1, NGP), lambda i: (0, 0))],
        out_specs=pl.BlockSpec((1, NGP), lambda i: (0, 0)),
        out_shape=jax.ShapeDtypeStruct((1, NGP), I32),
    )


def _rank_call(mp):
    ri, rj = 256, 512

    def body(kc_ref, bc_ref, kr_ref, br_ref, o_ref):
        i = pl.program_id(0)
        j = pl.program_id(1)

        @pl.when(j == 0)
        def _():
            o_ref[...] = jnp.zeros_like(o_ref)

        ii = i * ri + lax.broadcasted_iota(I32, (ri, 1), 0)
        jj = j * rj + lax.broadcasted_iota(I32, (1, rj), 1)
        kc = kc_ref[...]
        kr = kr_ref[...]
        same = bc_ref[...] == br_ref[...]
        cmp = (kr > kc) | ((kr == kc) & (jj < ii))
        o_ref[...] += jnp.sum((same & cmp).astype(I32), axis=1, keepdims=True)

    return pl.pallas_call(
        body,
        grid=(mp // ri, mp // rj),
        in_specs=[
            pl.BlockSpec((ri, 1), lambda i, j: (i, 0)),
            pl.BlockSpec((ri, 1), lambda i, j: (i, 0)),
            pl.BlockSpec((1, rj), lambda i, j: (0, j)),
            pl.BlockSpec((1, rj), lambda i, j: (0, j)),
        ],
        out_specs=pl.BlockSpec((ri, 1), lambda i, j: (i, 0)),
        out_shape=jax.ShapeDtypeStruct((mp, 1), I32),
    )


def _mapping_call(mp, mn):
    r = 512
    trash0 = mn - 128

    def body(rank_ref, b_ref, fit_ref, kk_ref, midx_ref, map_ref, bv_ref,
             fv_ref):
        i = pl.program_id(0)
        gi = lax.broadcasted_iota(I32, (1, NGP), 1)
        kkv = kk_ref[...]
        b = b_ref[...]
        rank = rank_ref[...]
        kk_i = jnp.sum(jnp.where(b == gi, kkv, 0), axis=1, keepdims=True)
        off_i = jnp.sum(jnp.where(gi < b, kkv, 0), axis=1, keepdims=True)
        keep = (rank < kk_i) & (b < NG)
        mapping = jnp.where(keep, off_i + rank, -1)
        map_ref[...] = mapping
        rowid = i * r + lax.broadcasted_iota(I32, (r, 1), 0)
        midx_ref[...] = jnp.where(keep, mapping, trash0 + (rowid & 127))
        bv_ref[...] = jnp.where(keep, b, NG)
        fv_ref[...] = jnp.where(keep, fit_ref[...], 0.0)

    return pl.pallas_call(
        body,
        grid=(mp // r,),
        in_specs=[
            pl.BlockSpec((r, 1), lambda i: (i, 0)),
            pl.BlockSpec((r, 1), lambda i: (i, 0)),
            pl.BlockSpec((r, 1), lambda i: (i, 0)),
            pl.BlockSpec((1, NGP), lambda i: (0, 0)),
        ],
        out_specs=[pl.BlockSpec((r, 1), lambda i: (i, 0))] * 4,
        out_shape=[jax.ShapeDtypeStruct((mp, 1), I32),
                   jax.ShapeDtypeStruct((mp, 1), I32),
                   jax.ShapeDtypeStruct((mp, 1), I32),
                   jax.ShapeDtypeStruct((mp, 1), F32)],
    )


def _readout_call(mp):
    r = 512

    def body(x_ref, f_ref, b_ref, sum_ref, mx_ref):
        i = pl.program_id(0)
        xv = x_ref[...] * f_ref[...]
        b = b_ref[...]
        gi = lax.broadcasted_iota(I32, (1, NGP), 1)
        oh = (b == gi).astype(F32)

        @pl.when(i == 0)
        def _():
            sum_ref[...] = jnp.zeros_like(sum_ref)
            mx_ref[...] = jnp.full_like(mx_ref, -jnp.inf)

        sum_ref[...] += lax.dot_general(oh, xv, (((0,), (0,)), ((), ())),
                                        preferred_element_type=F32)
        for g in range(NG):
            row = jnp.max(jnp.where(b == g, xv, -jnp.inf), axis=0,
                          keepdims=True)
            mx_ref[g:g + 1, :] = jnp.maximum(mx_ref[g:g + 1, :], row)

    return pl.pallas_call(
        body,
        grid=(mp // r,),
        in_specs=[
            pl.BlockSpec((r, 512), lambda i: (i, 0)),
            pl.BlockSpec((r, 1), lambda i: (i, 0)),
            pl.BlockSpec((r, 1), lambda i: (i, 0)),
        ],
        out_specs=[pl.BlockSpec((NGP, 512), lambda i: (0, 0)),
                   pl.BlockSpec((NGP, 512), lambda i: (0, 0))],
        out_shape=[jax.ShapeDtypeStruct((NGP, 512), F32),
                   jax.ShapeDtypeStruct((NGP, 512), F32)],
    )


def _head_call():
    def bn(t, g, b, rm):
        mu = jnp.sum(t * rm, axis=0, keepdims=True) / float(NG)
        var = jnp.sum(rm * (t - mu) ** 2, axis=0, keepdims=True) / float(NG)
        return jnp.maximum(g * (t - mu) * lax.rsqrt(var + 1e-5) + b, 0.0)

    def body(s1, m1, k1, s2, m2, k2, s3, m3, k3, wa, wb, l1b, g4, b4, w2, l2b,
             g5, b5, w3, l3b, o_ref):
        rm = (lax.broadcasted_iota(I32, (NGP, 1), 0) < NG).astype(F32)

        def xl(s_ref, m_ref, k_ref):
            kc = k_ref[...].astype(F32)
            mean = s_ref[...] / jnp.clip(kc, 1.0)
            mx = jnp.where(kc > 0, m_ref[...], 0.0)
            return mx, mean

        x1m, x1a = xl(s1, m1, k1)
        x2m, x2a = xl(s2, m2, k2)
        x3m, x3a = xl(s3, m3, k3)
        mxz = (x1m + x2m + x3m) / 3.0
        mnz = (x1a + x2a + x3a) / 3.0
        t = (jnp.dot(mxz, wa[...], preferred_element_type=F32)
             + jnp.dot(mnz, wb[...], preferred_element_type=F32) + l1b[...])
        t = bn(t, g4[...], b4[...], rm)
        u = jnp.dot(t, w2[...], preferred_element_type=F32) + l2b[...]
        u = bn(u, g5[...], b5[...], rm)
        logits = jnp.dot(u, w3[...], preferred_element_type=F32) + l3b[...]
        lanem = lax.broadcasted_iota(I32, (1, 128), 1) < 10
        mm = jnp.max(jnp.where(lanem, logits, -jnp.inf), axis=1, keepdims=True)
        lse = jnp.log(jnp.sum(jnp.where(lanem, jnp.exp(logits - mm), 0.0),
                              axis=1, keepdims=True)) + mm
        o_ref[...] = logits - lse

    full = lambda shape: pl.BlockSpec(shape, lambda: tuple(0 for _ in shape))
    in_specs = []
    for _ in range(3):
        in_specs += [full((NGP, 512)), full((NGP, 512)), full((NGP, 1))]
    in_specs += [full((512, 512)), full((512, 512)), full((1, 512)),
                 full((1, 512)), full((1, 512)), full((512, 256)),
                 full((1, 256)), full((1, 256)), full((1, 256)),
                 full((256, 128)), full((1, 128))]
    return pl.pallas_call(
        body,
        grid=(),
        in_specs=in_specs,
        out_specs=full((NGP, 128)),
        out_shape=jax.ShapeDtypeStruct((NGP, 128), F32),
    )


def _vfill(ref, n, val, dtype):
    for k in range(n // 16):
        ref[pl.ds(k * 16, 16)] = jnp.full((16,), val, dtype)


_MESH = None


def _mesh():
    global _MESH
    if _MESH is None:
        _MESH = plsc.VectorSubcoreMesh(core_axis_name="c", subcore_axis_name="s")
    return _MESH


def _sck_deg_call(mp):
    sl = mp // NT
    ept = E2 // NT
    nb = ept // 128

    @functools.partial(
        pl.kernel,
        out_type=jax.ShapeDtypeStruct((mp,), F32),
        mesh=_mesh(),
        scratch_types=[
            pltpu.VMEM((128,), I32),
            pltpu.VMEM((128,), F32),
            pltpu.VMEM_SHARED((mp,), F32),
        ],
    )
    def k(dst_hbm, zeros_hbm, out_hbm, idxv, onesv, acc):
        c = lax.axis_index("c")
        s = lax.axis_index("s")

        @pl.when(c == 0)
        def _():
            pltpu.sync_copy(zeros_hbm.at[pl.ds(s * sl, sl)],
                            acc.at[pl.ds(s * sl, sl)])
            _vfill(onesv, 128, 1.0, F32)
            plsc.subcore_barrier()

            def blk(b, carry):
                base = s * ept + b * 128
                pltpu.sync_copy(dst_hbm.at[pl.ds(base, 128)], idxv)
                pltpu.sync_copy(onesv, acc.at[idxv], add=True)
                return carry

            lax.fori_loop(0, nb, blk, 0)
            plsc.subcore_barrier()
            pltpu.sync_copy(acc.at[pl.ds(s * sl, sl)],
                            out_hbm.at[pl.ds(s * sl, sl)])

    return k


def _sck_conv_call(mp):
    sl = mp // NT
    ept = E2 // NT
    nb = ept // 128

    @functools.partial(
        pl.kernel,
        out_type=jax.ShapeDtypeStruct((4, mp, 128), F32),
        mesh=_mesh(),
        scratch_types=[
            pltpu.VMEM((128,), I32),
            pltpu.VMEM((128,), I32),
            pltpu.VMEM((128,), I32),
            pltpu.VMEM((128,), I32),
            pltpu.VMEM((128, 128), F32),
            pltpu.VMEM((128, 128), F32),
            pltpu.VMEM_SHARED((mp, 128), F32),
            pltpu.SemaphoreType.DMA,
            pltpu.SemaphoreType.DMA,
        ],
    )
    def k(hq_hbm, src_hbm, dst_hbm, zeros_hbm, out_hbm, sidx0, didx0, sidx1,
          didx1, rows0, rows1, acc, sem0, sem1):
        c = lax.axis_index("c")
        s = lax.axis_index("s")

        def one_chunk(q):
            pltpu.sync_copy(zeros_hbm.at[pl.ds(s * sl, sl)],
                            acc.at[pl.ds(s * sl, sl)])
            plsc.subcore_barrier()

            def start(b, sidx, didx, rows, sem):
                base = s * ept + b * 128
                pltpu.sync_copy(src_hbm.at[pl.ds(base, 128)], sidx)
                pltpu.sync_copy(dst_hbm.at[pl.ds(base, 128)], didx)
                pltpu.async_copy(hq_hbm.at[q].at[sidx], rows, sem)

            def drain_scatter(sidx, didx, rows, sem):
                pltpu.make_async_copy(hq_hbm.at[q].at[sidx], rows, sem).wait()
                pltpu.sync_copy(rows, acc.at[didx], add=True)

            start(0, sidx0, didx0, rows0, sem0)

            def pair(p, carry):
                b0 = 2 * p
                start(b0 + 1, sidx1, didx1, rows1, sem1)
                drain_scatter(sidx0, didx0, rows0, sem0)

                @pl.when(b0 + 2 < nb)
                def _():
                    start(b0 + 2, sidx0, didx0, rows0, sem0)

                drain_scatter(sidx1, didx1, rows1, sem1)
                return carry

            lax.fori_loop(0, nb // 2, pair, 0)
            if nb % 2:
                drain_scatter(sidx0, didx0, rows0, sem0)
            plsc.subcore_barrier()
            pltpu.sync_copy(acc.at[pl.ds(s * sl, sl)],
                            out_hbm.at[q].at[pl.ds(s * sl, sl)])
            plsc.subcore_barrier()

        for ci in range(2):
            @pl.when(c == ci)
            def _():
                one_chunk(2 * ci)
                one_chunk(2 * ci + 1)

    return k


def _sck_fitness_call(mp, acoefs):
    sl = mp // NT
    ept = E2 // NT
    nb = ept // 128

    @functools.partial(
        pl.kernel,
        out_type=jax.ShapeDtypeStruct((mp,), F32),
        mesh=_mesh(),
        scratch_types=[
            pltpu.VMEM((128,), I32),
            pltpu.VMEM((128,), I32),
            pltpu.VMEM((128,), F32),
            pltpu.VMEM((128,), I32),
            pltpu.VMEM((128,), I32),
            pltpu.VMEM((128,), F32),
            pltpu.SemaphoreType.DMA,
            pltpu.VMEM((sl,), F32),
            pltpu.VMEM((sl,), F32),
            pltpu.VMEM((sl,), F32),
            pltpu.VMEM((sl,), F32),
            pltpu.VMEM((sl,), F32),
            pltpu.VMEM((sl,), F32),
            pltpu.VMEM((64,), F32),
            pltpu.VMEM_SHARED((mp,), F32),
            pltpu.VMEM_SHARED((mp,), F32),
            pltpu.SemaphoreType.DMA,
        ],
    )
    def k(s_hbm, dinv_hbm, src_hbm, dst_hbm, coefb_hbm, zeros_hbm, out_hbm,
          sidx, didx, vals, sidx1, didx1, vals1, sem1, dsl, stage, pm2, pm1,
          pk, osl, cvec, upub, acc, sem):
        c = lax.axis_index("c")
        s = lax.axis_index("s")

        @pl.when(c == 0)
        def _():
            base = s * sl
            pltpu.sync_copy(dinv_hbm.at[pl.ds(base, sl)], dsl)
            pltpu.sync_copy(s_hbm.at[pl.ds(base, sl)], pm2)
            pltpu.sync_copy(coefb_hbm, cvec)

            def coef(kc):
                return cvec[pl.ds(kc * 16, 16)]

            def spmv(v_ref):
                for t in range(sl // 16):
                    d16 = pl.ds(t * 16, 16)
                    stage[d16] = dsl[d16] * v_ref[d16]
                pltpu.sync_copy(stage, upub.at[pl.ds(base, sl)])
                pltpu.sync_copy(zeros_hbm.at[pl.ds(base, sl)],
                                acc.at[pl.ds(base, sl)])
                plsc.subcore_barrier()

                def start(b, si, di, va, se):
                    eb = s * ept + b * 128
                    pltpu.sync_copy(src_hbm.at[pl.ds(eb, 128)], si)
                    pltpu.sync_copy(dst_hbm.at[pl.ds(eb, 128)], di)
                    pltpu.async_copy(upub.at[si], va, se)

                def drain(si, di, va, se):
                    pltpu.make_async_copy(upub.at[si], va, se).wait()
                    pltpu.sync_copy(va, acc.at[di], add=True)

                start(0, sidx, didx, vals, sem)

                def pair(p, carry):
                    b0 = 2 * p
                    start(b0 + 1, sidx1, didx1, vals1, sem1)
                    drain(sidx, didx, vals, sem)

                    @pl.when(b0 + 2 < nb)
                    def _():
                        start(b0 + 2, sidx, didx, vals, sem)

                    drain(sidx1, didx1, vals1, sem1)
                    return carry

                lax.fori_loop(0, nb // 2, pair, 0)
                if nb % 2:
                    drain(sidx, didx, vals, sem)
                plsc.subcore_barrier()
                pltpu.sync_copy(acc.at[pl.ds(base, sl)], stage)

            c0 = coef(0)
            for t in range(sl // 16):
                d16 = pl.ds(t * 16, 16)
                osl[d16] = c0 * pm2[d16]
            spmv(pm2)
            c1c = coef(1)
            for t in range(sl // 16):
                d16 = pl.ds(t * 16, 16)
                av = dsl[d16] * stage[d16] + dsl[d16] * dsl[d16] * pm2[d16]
                pm1[d16] = 2.0 * av
                osl[d16] = osl[d16] + c1c * pm1[d16]
            for kk_, (cc1, cc2, cc3, cc4) in enumerate(acoefs):
                spmv(pm1)
                ck = coef(kk_ + 2)
                for t in range(sl // 16):
                    d16 = pl.ds(t * 16, 16)
                    av = dsl[d16] * stage[d16] + dsl[d16] * dsl[d16] * pm1[d16]
                    pkv = (cc2 * pm1[d16] + cc3 * av - cc4 * pm2[d16]) / cc1
                    pk[d16] = pkv
                    osl[d16] = osl[d16] + ck * pkv
                for t in range(sl // 16):
                    d16 = pl.ds(t * 16, 16)
                    pm2[d16] = pm1[d16]
                    pm1[d16] = pk[d16]
            pltpu.sync_copy(osl, out_hbm.at[pl.ds(base, sl)])

    return k


def _sck_pool_call(mp, mn, phases=(0, 1, 2, 3)):
    sl = mp // NT
    sln = mn // NT
    ept = E2 // NT
    nb = ept // 128
    trash0 = mn - 128

    out_type = [
        jax.ShapeDtypeStruct((mn,), I32),
        jax.ShapeDtypeStruct((mn,), I32),
        jax.ShapeDtypeStruct((mn,), F32),
        jax.ShapeDtypeStruct((mn, 512), F32),
        jax.ShapeDtypeStruct((E2,), I32),
        jax.ShapeDtypeStruct((E2,), I32),
        jax.ShapeDtypeStruct((mn,), F32),
    ]

    @functools.partial(
        pl.kernel,
        out_type=out_type,
        mesh=_mesh(),
        scratch_types=[
            pltpu.VMEM((128,), I32),
            pltpu.VMEM((128,), I32),
            pltpu.VMEM((128,), I32),
            pltpu.VMEM((128,), F32),
            pltpu.VMEM((128,), F32),
            pltpu.VMEM((128,), I32),
            pltpu.VMEM((128,), F32),
            pltpu.VMEM((64,), I32),
            pltpu.VMEM((64, 512), F32),
            pltpu.VMEM((128,), I32),
            pltpu.VMEM((128,), I32),
            pltpu.VMEM((128,), I32),
            pltpu.VMEM((128,), I32),
            pltpu.VMEM((128,), I32),
            pltpu.VMEM((128,), I32),
            pltpu.SemaphoreType.DMA,
            pltpu.VMEM_SHARED((mn,), F32),
            pltpu.SemaphoreType.DMA,
        ],
    )
    def k(map_hbm, midx_hbm, bval_hbm, fval_hbm, iota_hbm, h_hbm, src_hbm,
          dst_hbm, zeros_hbm, minv_hbm, bnew_hbm, fnew_hbm, xraw_hbm,
          srcn_hbm, dstn_hbm, deg_hbm, ia, ib, iv, fv, ones, dbi, dbf, ri,
          rows, smv, dmv, ia2, ib2, smv2, dmv2, sem2, dacc, sem):
        c = lax.axis_index("c")
        s = lax.axis_index("s")

        @pl.when(c == 0)
        def _():
            if 0 not in phases:
                return
            _vfill(dbi, 128, NG, I32)
            _vfill(dbf, 128, 0.0, F32)
            for bidx in range(sln // 128):
                b0 = s * sln + bidx * 128
                pltpu.sync_copy(dbf, fnew_hbm.at[pl.ds(b0, 128)])
                pltpu.sync_copy(dbi, bnew_hbm.at[pl.ds(b0, 128)])
            _vfill(dbi, 128, 0, I32)
            for bidx in range(sln // 128):
                b0 = s * sln + bidx * 128
                pltpu.sync_copy(dbi, minv_hbm.at[pl.ds(b0, 128)])
            plsc.subcore_barrier()

            if 1 not in phases:
                return
            nbk = mp // 128

            def scat(b0):
                pltpu.sync_copy(midx_hbm.at[pl.ds(b0, 128)], ia)
                pltpu.sync_copy(iota_hbm.at[pl.ds(b0, 128)], iv)
                pltpu.sync_copy(iv, minv_hbm.at[ia])
                pltpu.sync_copy(bval_hbm.at[pl.ds(b0, 128)], iv)
                pltpu.sync_copy(iv, bnew_hbm.at[ia])
                pltpu.sync_copy(fval_hbm.at[pl.ds(b0, 128)], fv)
                pltpu.sync_copy(fv, fnew_hbm.at[ia])

            for j in range((nbk + NT - 1) // NT):
                bi = s + j * NT
                if (j + 1) * NT <= nbk:
                    scat(bi * 128)
                else:
                    @pl.when(bi < nbk)
                    def _():
                        scat(bi * 128)
            plsc.subcore_barrier()

            if 2 not in phases:
                return

            def gat(r0):
                pltpu.sync_copy(minv_hbm.at[pl.ds(r0, 64)], ri)
                pltpu.async_copy(h_hbm.at[ri], rows, sem).wait()
                pltpu.sync_copy(rows, xraw_hbm.at[pl.ds(r0, 64)])

            for bidx in range(sln // 64):
                gat(s * sln + bidx * 64)

        @pl.when(c == 1)
        def _():
            if 3 not in phases:
                return
            _vfill(ones, 128, 1.0, F32)
            pltpu.sync_copy(zeros_hbm.at[pl.ds(s * sln, sln)],
                            dacc.at[pl.ds(s * sln, sln)])
            plsc.subcore_barrier()
            io16 = lax.iota(I32, 16)

            def start(b, pia, pib, psm, pdm, se):
                eb = s * ept + b * 128
                pltpu.sync_copy(src_hbm.at[pl.ds(eb, 128)], pia)
                pltpu.sync_copy(dst_hbm.at[pl.ds(eb, 128)], pib)
                pltpu.async_copy(map_hbm.at[pia], psm, se)
                pltpu.async_copy(map_hbm.at[pib], pdm, se)

            def drain(b, pia, pib, psm, pdm, se):
                eb = s * ept + b * 128
                pltpu.make_async_copy(map_hbm.at[pia], psm, se).wait()
                pltpu.make_async_copy(map_hbm.at[pib], pdm, se).wait()
                for t in range(8):
                    d16 = pl.ds(t * 16, 16)
                    sm = psm[d16]
                    dm = pdm[d16]
                    keep = (sm >= 0) & (dm >= 0)
                    eidx = eb + t * 16 + io16
                    pia[d16] = jnp.where(keep, sm, eidx & 1023)
                    pib[d16] = jnp.where(keep, dm, trash0 + (eidx & 127))
                pltpu.sync_copy(pia, srcn_hbm.at[pl.ds(eb, 128)])
                pltpu.sync_copy(pib, dstn_hbm.at[pl.ds(eb, 128)])
                pltpu.sync_copy(ones, dacc.at[pib], add=True)

            start(0, ia, ib, smv, dmv, sem)

            def pair(p, carry):
                b0 = 2 * p
                start(b0 + 1, ia2, ib2, smv2, dmv2, sem2)
                drain(b0, ia, ib, smv, dmv, sem)

                @pl.when(b0 + 2 < nb)
                def _():
                    start(b0 + 2, ia, ib, smv, dmv, sem)

                drain(b0 + 1, ia2, ib2, smv2, dmv2, sem2)
                return carry

            lax.fori_loop(0, nb // 2, pair, 0)
            if nb % 2:
                drain(nb - 1, ia, ib, smv, dmv, sem)
            plsc.subcore_barrier()
            pltpu.sync_copy(dacc.at[pl.ds(s * sln, sln)],
                            deg_hbm.at[pl.ds(s * sln, sln)])

    return k


def _jacobi_recur_consts():
    a, b = 1.0, 1.0
    out = []
    for k in range(2, 4):
        c1 = 2.0 * k * (k + a + b) * (2 * k + a + b - 2)
        c2 = (2 * k + a + b - 1) * (a * a - b * b)
        c3 = (2 * k + a + b - 2) * (2 * k + a + b - 1) * (2 * k + a + b)
        c4 = 2.0 * (k + a - 1) * (k + b - 1) * (2 * k + a + b)
        out.append((c1, c2, c3, c4))
    return tuple(out)


def _layer(li, x, rowscale, batch2d, src, dst, deg1d, W, bias, bng, bnb,
           poolw, poolc, counts, zeros1d, zeros2d):
    mp, mn = MP[li], MP[li + 1]
    deg2d = deg1d.reshape(mp, 1)
    dinv2d, dinv1d2 = _dinv_call(mp)(deg2d)
    dinv1d = dinv1d2.reshape(mp)
    if rowscale is None:
        hpre, hq = _mm_pre_call(mp, W.shape[0], False)(x, W, dinv2d)
    else:
        hpre, hq = _mm_pre_call(mp, W.shape[0], True)(x, W, dinv2d, rowscale)
    S = _sck_conv_call(mp)(hq, src, dst, zeros2d[:mp])
    z, st = _bnormA_call(mp)(S, hpre, dinv2d, batch2d, bias.reshape(1, 512))
    h, s2d = _bnormB_call(mp)(z, st, bng.reshape(1, 512), bnb.reshape(1, 512),
                              poolw.reshape(512, 1))
    coefb = jnp.repeat(poolc, 16)
    fpre = _sck_fitness_call(mp, _jacobi_recur_consts())(
        s2d.reshape(mp), dinv1d, src, dst, coefb, zeros1d[:mp])
    fit2d, key2d = _tanh_key_call(mp)(fpre.reshape(mp, 1))
    kk = _kk_call()(counts)
    keyrow = key2d.reshape(1, mp)
    batchrow = batch2d.reshape(1, mp)
    rank2d = _rank_call(mp)(key2d, batch2d, keyrow, batchrow)
    midx, mapping, bval, fval = _mapping_call(mp, mn)(rank2d, batch2d, fit2d,
                                                      kk)
    iota1d = jnp.arange(mp, dtype=I32)
    minv, bnew, fnew, xraw, srcn, dstn, degn = _sck_pool_call(mp, mn)(
        mapping.reshape(mp), midx.reshape(mp), bval.reshape(mp),
        fval.reshape(mp), iota1d, h, src, dst, zeros1d[:mn])
    fnew2d = fnew.reshape(mn, 1)
    bnew2d = bnew.reshape(mn, 1)
    ro_sum, ro_mx = _readout_call(mn)(xraw, fnew2d, bnew2d)
    return (xraw, fnew2d, bnew2d, srcn, dstn, degn, kk, ro_sum, ro_mx)


def kernel(x, edge_index, batch, W1, b1, bn1_g, bn1_b, pool1_w, pool1_c, W2,
           b2, bn2_g, bn2_b, pool2_w, pool2_c, W3, b3, bn3_g, bn3_b, pool3_w,
           pool3_c, lin1_W, lin1_b, bn4_g, bn4_b, lin2_W, lin2_b, bn5_g,
           bn5_b, lin3_W, lin3_b):
    mp0 = MP[0]
    x_p = jnp.pad(x, ((0, mp0 - NREAL), (0, 0)))
    batch2d = jnp.pad(batch.astype(I32), (0, mp0 - NREAL),
                      constant_values=NG).reshape(mp0, 1)
    epad = E2 - EREAL
    ep = jnp.arange(epad, dtype=I32)
    src0 = jnp.concatenate([edge_index[0].astype(I32), ep & 1023])
    dst0 = jnp.concatenate([edge_index[1].astype(I32),
                            (mp0 - 128) + (ep & 127)])
    zeros1d = jnp.zeros((mp0,), F32)
    zeros2d = jnp.zeros((mp0, 128), F32)

    deg0 = _sck_deg_call(mp0)(dst0, zeros1d)
    counts1 = _counts_call(mp0)(batch2d)

    (x2, f2, b2d, src1, dst1, deg1, kk1, s1m, x1mx) = _layer(
        0, x_p, None, batch2d, src0, dst0, deg0, W1, b1, bn1_g, bn1_b,
        pool1_w, pool1_c, counts1, zeros1d, zeros2d)
    (x3, f3, b3d, src2, dst2, deg2, kk2, s2m, x2mx) = _layer(
        1, x2, f2, b2d, src1, dst1, deg1, W2, b2, bn2_g, bn2_b, pool2_w,
        pool2_c, kk1, zeros1d, zeros2d)
    (_x4, _f4, _b4d, _s, _d, _dg, kk3, s3m, x3mx) = _layer(
        2, x3, f3, b3d, src2, dst2, deg2, W3, b3, bn3_g, bn3_b, pool3_w,
        pool3_c, kk2, zeros1d, zeros2d)

    kkc1 = kk1.reshape(NGP, 1).astype(F32)
    kkc2 = kk2.reshape(NGP, 1).astype(F32)
    kkc3 = kk3.reshape(NGP, 1).astype(F32)
    out = _head_call()(
        s1m, x1mx, kkc1, s2m, x2mx, kkc2, s3m, x3mx, kkc3,
        lin1_W[:512], lin1_W[512:], lin1_b.reshape(1, 512),
        bn4_g.reshape(1, 512), bn4_b.reshape(1, 512), lin2_W,
        lin2_b.reshape(1, 256), bn5_g.reshape(1, 256), bn5_b.reshape(1, 256),
        jnp.pad(lin3_W, ((0, 0), (0, 118))),
        jnp.pad(lin3_b, (0, 118)).reshape(1, 128))
    return out[:NG, :10]
